# baseline (speedup 1.0000x reference)
import functools

import jax
import jax.numpy as jnp
from jax import lax
from jax.experimental import pallas as pl
from jax.experimental.pallas import tpu as pltpu
from jax.experimental.pallas import tpu_sc as plsc

B = 8
N_LIDAR, N_PROP, N_GLOB = 16384, 8192, 8192
N_TOT = N_LIDAR + N_PROP + N_GLOB
D = 128
KEEP = 1000
CHUNK = 2048
N_CH_L = N_LIDAR // CHUNK
N_CH_P = N_PROP // CHUNK
N_CH_G = N_GLOB // CHUNK
N_CH = N_CH_L + N_CH_P + N_CH_G


def _score_body(emb_ref, w1_ref, b1_ref, w2_ref, b2_ref,
                lq_ref, ls_ref, pq_ref, ps_ref, gq_ref, gs_ref,
                ks_ref, kl_ref, as_ref):
    g = pl.program_id(1)
    is_l = g < N_CH_L
    is_p = jnp.logical_and(g >= N_CH_L, g < N_CH_L + N_CH_P)
    q = jnp.where(is_l, lq_ref[0, 0], jnp.where(is_p, pq_ref[0, 0], gq_ref[0, 0]))
    s = jnp.where(is_l, ls_ref[0, 0, 0], jnp.where(is_p, ps_ref[0, 0, 0], gs_ref[0, 0, 0]))
    e = jnp.where(is_l, emb_ref[0:1, :], jnp.where(is_p, emb_ref[1:2, :], emb_ref[2:3, :]))
    aq = q + e
    feat = jnp.concatenate([aq, s[:, None]], axis=1)
    h = jnp.maximum(jnp.dot(feat, w1_ref[...], preferred_element_type=jnp.float32)
                    + b1_ref[0:1, :], 0.0)
    logits = jnp.dot(h, w2_ref[...], preferred_element_type=jnp.float32)[:, 0] + b2_ref[0, 0]
    ks_ref[0, 0, 0, :] = logits + s
    kl_ref[0, 0, 0, :] = logits
    as_ref[0, 0, 0, :] = s


@functools.partial(jax.jit, static_argnames=("interpret",))
def _score_call(lq, ls, pq, ps, gq, gs, emb, w1, b1, w2, b2, interpret=False):
    ls3 = ls.reshape(B, N_CH_L, 1, CHUNK)
    ps3 = ps.reshape(B, N_CH_P, 1, CHUNK)
    gs3 = gs.reshape(B, N_CH_G, 1, CHUNK)
    grid = (B, N_CH)

    def qmap(lo, hi):
        return lambda b, g: (b, jnp.clip(g - lo, 0, hi - lo - 1), 0, 0)

    def smap(lo, hi):
        return lambda b, g: (b, jnp.clip(g - lo, 0, hi - lo - 1), 0, 0)

    out = pl.pallas_call(
        _score_body,
        grid=grid,
        in_specs=[
            pl.BlockSpec((3, D), lambda b, g: (0, 0)),
            pl.BlockSpec((D + 1, D), lambda b, g: (0, 0)),
            pl.BlockSpec((1, D), lambda b, g: (0, 0)),
            pl.BlockSpec((D, 1), lambda b, g: (0, 0)),
            pl.BlockSpec((1, 1), lambda b, g: (0, 0)),
            pl.BlockSpec((1, 1, CHUNK, D), qmap(0, N_CH_L)),
            pl.BlockSpec((1, 1, 1, CHUNK), smap(0, N_CH_L)),
            pl.BlockSpec((1, 1, CHUNK, D), qmap(N_CH_L, N_CH_L + N_CH_P)),
            pl.BlockSpec((1, 1, 1, CHUNK), smap(N_CH_L, N_CH_L + N_CH_P)),
            pl.BlockSpec((1, 1, CHUNK, D), qmap(N_CH_L + N_CH_P, N_CH)),
            pl.BlockSpec((1, 1, 1, CHUNK), smap(N_CH_L + N_CH_P, N_CH)),
        ],
        out_specs=[
            pl.BlockSpec((1, 1, 1, CHUNK), lambda b, g: (b, g, 0, 0)),
            pl.BlockSpec((1, 1, 1, CHUNK), lambda b, g: (b, g, 0, 0)),
            pl.BlockSpec((1, 1, 1, CHUNK), lambda b, g: (b, g, 0, 0)),
        ],
        out_shape=[jax.ShapeDtypeStruct((B, N_CH, 1, CHUNK), jnp.float32)] * 3,
        interpret=interpret,
    )(emb, w1, b1.reshape(1, D), w2, b2.reshape(1, 1),
      lq.reshape(B, N_CH_L, CHUNK, D), ls3,
      pq.reshape(B, N_CH_P, CHUNK, D), ps3,
      gq.reshape(B, N_CH_G, CHUNK, D), gs3)
    ks, kl, asc = (o.reshape(B, N_TOT) for o in out)
    return ks, kl, asc


N_VREG = N_TOT // 16
PADK = 1024


def _u32(x):
    return x.astype(jnp.uint32)


def _vperm(x, perm):
    return lax.gather(
        x, perm[:, None],
        lax.GatherDimensionNumbers(offset_dims=(), collapsed_slice_dims=(0,),
                                   start_index_map=(0,)),
        (1,), mode=lax.GatherScatterMode.PROMISE_IN_BOUNDS)


def _topk_body(ks_hbm, asc_hbm, kl_hbm, out_hbm, outs_hbm, outl_hbm,
               outsrc_hbm, key_v, asc_v, kl_v, hist_v, gtk_v, gti_v, eqi_v,
               srtk_v, srti_v, gs_v, gl_v, gsrc_v, sem):
    c = lax.axis_index("c")
    s = lax.axis_index("s")
    wid = s * 2 + c

    @pl.when(wid < B)
    def _run():
        b = wid
        lanes = lax.iota(jnp.int32, 16)
        pltpu.sync_copy(ks_hbm.at[b], key_v)
        pltpu.sync_copy(asc_hbm.at[b], asc_v)
        pltpu.sync_copy(kl_hbm.at[b], kl_v)

        def _clear_hist(i, _):
            hist_v[pl.ds(i * 16, 16)] = jnp.zeros((16,), jnp.int32)
            return 0

        ones = jnp.ones((16,), jnp.int32)

        lax.fori_loop(0, 256, _clear_hist, 0)

        def _p1(i, _):
            x = key_v[pl.ds(i * 16, 16)]
            u = x.astype(jnp.uint32)
            neg = u >> 31
            m = (jnp.uint32(0) - neg) | jnp.uint32(0x80000000)
            k = u ^ m
            key_v[pl.ds(i * 16, 16)] = k.astype(jnp.int32)
            d = (k >> 24).astype(jnp.int32)
            plsc.addupdate_scatter(hist_v, [d * 16 + lanes], ones)
            return 0

        lax.fori_loop(0, N_VREG, _p1, 0)

        def _scan_bins(krem):
            def bscan(i, carry):
                sel, above, cum, found = carry
                bin_ = 255 - i
                cnt = jnp.sum(hist_v[pl.ds(bin_ * 16, 16)])
                newcum = cum + cnt
                hit = jnp.logical_and(jnp.logical_not(found), newcum >= krem)
                sel = jnp.where(hit, bin_, sel)
                above = jnp.where(hit, cum, above)
                return sel, above, newcum, jnp.logical_or(found, hit)

            sel, above, _, _ = lax.fori_loop(
                0, 256, bscan, (jnp.int32(0), jnp.int32(0), jnp.int32(0),
                                jnp.bool_(False)))
            return sel, above

        krem = jnp.int32(KEEP)
        sel, above = _scan_bins(krem)
        prefix = _u32(sel)
        krem = krem - above

        for shift in (16, 8, 0):
            lax.fori_loop(0, 256, _clear_hist, 0)
            pfx = prefix

            def _pp(i, _, shift=shift, pfx=pfx):
                k = key_v[pl.ds(i * 16, 16)].astype(jnp.uint32)
                msk = (k >> (shift + 8)) == pfx
                d = ((k >> shift) & jnp.uint32(0xFF)).astype(jnp.int32)
                plsc.addupdate_scatter(hist_v, [d * 16 + lanes], ones, mask=msk)
                return 0

            lax.fori_loop(0, N_VREG, _pp, 0)
            sel, above = _scan_bins(krem)
            prefix = (prefix << 8) | _u32(sel)
            krem = krem - above

        t = prefix

        def _comp(i, carry):
            pg, pe = carry
            k = key_v[pl.ds(i * 16, 16)].astype(jnp.uint32)
            idxv = i * 16 + lanes
            m_gt = k > t
            m_eq = k == t
            plsc.store_compressed(gtk_v.at[pl.ds(pg, 16)], k.astype(jnp.int32), mask=m_gt)
            plsc.store_compressed(gti_v.at[pl.ds(pg, 16)], idxv, mask=m_gt)

            @pl.when(pe < PADK)
            def _():
                plsc.store_compressed(eqi_v.at[pl.ds(pe, 16)], idxv, mask=m_eq)

            pg = pg + jnp.sum(m_gt.astype(jnp.int32))
            pe = pe + jnp.sum(m_eq.astype(jnp.int32))
            return pg, pe

        n_gt, _ = lax.fori_loop(0, N_VREG, _comp, (jnp.int32(0), jnp.int32(0)))

        def _init(i, _):
            srtk_v[pl.ds(i * 16, 16)] = jnp.zeros((16,), jnp.uint32)
            srti_v[pl.ds(i * 16, 16)] = jnp.full((16,), 0x7FFFFFFF, jnp.int32)
            return 0

        lax.fori_loop(0, PADK // 16, _init, 0)

        def _cgt(i, _):
            pos = i * 16 + lanes
            m = pos < n_gt
            kk = gtk_v[pl.ds(i * 16, 16)].astype(jnp.uint32)
            ii = gti_v[pl.ds(i * 16, 16)]
            ok = srtk_v[pl.ds(i * 16, 16)]
            oi = srti_v[pl.ds(i * 16, 16)]
            srtk_v[pl.ds(i * 16, 16)] = jnp.where(m, kk, ok)
            srti_v[pl.ds(i * 16, 16)] = jnp.where(m, ii, oi)
            return 0

        lax.fori_loop(0, PADK // 16, _cgt, 0)

        def _ceq(j, _):
            jpos = j * 16 + lanes
            m = (n_gt + jpos) < KEEP
            e = eqi_v[pl.ds(j * 16, 16)]
            base = n_gt + j * 16
            ok = srtk_v[pl.ds(base, 16)]
            oi = srti_v[pl.ds(base, 16)]
            srtk_v[pl.ds(base, 16)] = jnp.where(m, jnp.full((16,), 1, jnp.uint32) * t, ok)
            srti_v[pl.ds(base, 16)] = jnp.where(m, e, oi)
            return 0

        lax.fori_loop(0, (KEEP + 15) // 16, _ceq, 0)

        perm_base = lanes

        def _lex_ge(ka, ia, kb, ib):
            return jnp.logical_or(
                ka > kb, jnp.logical_and(ka == kb, ia < ib))

        for size in (2, 4, 8, 16, 32, 64, 128, 256, 512, 1024):
            stride = size // 2
            while stride >= 16:
                w = stride // 16

                def _pair(p, _, w=w, size=size):
                    va = ((p & ~(w - 1)) << 1) | (p & (w - 1))
                    vb = va + w
                    dsc = ((va * 16) & size) == 0
                    ka = srtk_v[pl.ds(va * 16, 16)]
                    ia = srti_v[pl.ds(va * 16, 16)]
                    kb = srtk_v[pl.ds(vb * 16, 16)]
                    ib = srti_v[pl.ds(vb * 16, 16)]
                    ge = _lex_ge(ka, ia, kb, ib)
                    m = jnp.where(dsc, ge, jnp.logical_not(ge))
                    srtk_v[pl.ds(va * 16, 16)] = jnp.where(m, ka, kb)
                    srti_v[pl.ds(va * 16, 16)] = jnp.where(m, ia, ib)
                    srtk_v[pl.ds(vb * 16, 16)] = jnp.where(m, kb, ka)
                    srti_v[pl.ds(vb * 16, 16)] = jnp.where(m, ib, ia)
                    return 0

                lax.fori_loop(0, PADK // 32, _pair, 0)
                stride //= 2
            while stride >= 1:
                perm = perm_base ^ stride

                def _intra(v, _, stride=stride, size=size, perm=perm):
                    kk = srtk_v[pl.ds(v * 16, 16)]
                    ii = srti_v[pl.ds(v * 16, 16)]
                    kp = _vperm(kk, perm)
                    ip = _vperm(ii, perm)
                    low = (lanes & stride) == 0
                    dsc = ((v * 16 + lanes) & size) == 0
                    ge = _lex_ge(kk, ii, kp, ip)
                    cond = ge == (low == dsc)
                    srtk_v[pl.ds(v * 16, 16)] = jnp.where(cond, kk, kp)
                    srti_v[pl.ds(v * 16, 16)] = jnp.where(cond, ii, ip)
                    return 0

                lax.fori_loop(0, PADK // 16, _intra, 0)
                stride //= 2

        srti_v[pl.ds(KEEP, 16)] = lanes * 8
        srti_v[pl.ds(PADK - 16, 16)] = (lanes + 16) * 8

        def _gout(i, _):
            sidx = srti_v[pl.ds(i * 16, 16)]
            gs_v[pl.ds(i * 16, 16)] = plsc.load_gather(asc_v, [sidx])
            gl_v[pl.ds(i * 16, 16)] = plsc.load_gather(kl_v, [sidx])
            gsrc_v[pl.ds(i * 16, 16)] = (
                (sidx >= N_LIDAR).astype(jnp.int32)
                + (sidx >= N_LIDAR + N_PROP).astype(jnp.int32))
            return 0

        lax.fori_loop(0, PADK // 16, _gout, 0)
        pltpu.sync_copy(srti_v, out_hbm.at[b])
        pltpu.sync_copy(gs_v, outs_hbm.at[b])
        pltpu.sync_copy(gl_v, outl_hbm.at[b])
        pltpu.sync_copy(gsrc_v, outsrc_hbm.at[b])


@jax.jit
def _topk_call(ks, asc, kl):
    ksb = lax.bitcast_convert_type(ks, jnp.int32)
    mesh = plsc.VectorSubcoreMesh(core_axis_name="c", subcore_axis_name="s")
    f = pl.kernel(
        _topk_body,
        out_type=[
            jax.ShapeDtypeStruct((B, PADK), jnp.int32),
            jax.ShapeDtypeStruct((B, PADK), jnp.float32),
            jax.ShapeDtypeStruct((B, PADK), jnp.float32),
            jax.ShapeDtypeStruct((B, PADK), jnp.int32),
        ],
        mesh=mesh,
        compiler_params=pltpu.CompilerParams(needs_layout_passes=False),
        scratch_types=[
            pltpu.VMEM((N_TOT,), jnp.int32),
            pltpu.VMEM((N_TOT,), jnp.float32),
            pltpu.VMEM((N_TOT,), jnp.float32),
            pltpu.VMEM((256 * 16,), jnp.int32),
            pltpu.VMEM((PADK + 16,), jnp.int32),
            pltpu.VMEM((PADK + 16,), jnp.int32),
            pltpu.VMEM((PADK + 16,), jnp.int32),
            pltpu.VMEM((PADK,), jnp.uint32),
            pltpu.VMEM((PADK,), jnp.int32),
            pltpu.VMEM((PADK,), jnp.float32),
            pltpu.VMEM((PADK,), jnp.float32),
            pltpu.VMEM((PADK,), jnp.int32),
            pltpu.SemaphoreType.DMA,
        ],
    )
    return f(ksb, asc, kl)


ROWS_W = PADK // 4
HALF = 128


def _gather_body(top_hbm, lq_hbm, pq_hbm, gq_hbm, refs128_hbm, emb_hbm,
                 outq_hbm, outr4_hbm,
                 idx_v, rid_v, off_v, rid2_v, rows2_v, pos2_v,
                 r0_v, r1_v, r2_v, q0_v, q1_v, q2_v,
                 rrow_v, rbig_v, qrow_v, emb_v, sem):
    c = lax.axis_index("c")
    s = lax.axis_index("s")
    wid = s * 2 + c
    b = wid // 4
    part = wid % 4
    lanes = lax.iota(jnp.int32, 16)
    obase = b * PADK + part * ROWS_W

    pltpu.sync_copy(top_hbm.at[b, pl.ds(part * ROWS_W, ROWS_W)], idx_v)
    pltpu.sync_copy(emb_hbm, emb_v)

    def _dflt(j, _):
        safe = b * N_PROP + j * 16 + lanes
        dump = jnp.full((16,), b * PADK + PADK - 1, jnp.int32)
        r0_v[pl.ds(j * 16, 16)] = safe
        r1_v[pl.ds(j * 16, 16)] = safe
        r2_v[pl.ds(j * 16, 16)] = safe
        q0_v[pl.ds(j * 16, 16)] = dump
        q1_v[pl.ds(j * 16, 16)] = dump
        q2_v[pl.ds(j * 16, 16)] = dump
        return 0

    lax.fori_loop(0, ROWS_W // 16 + 1, _dflt, 0)

    def _split(j, carry):
        p0, p1, p2 = carry
        ix = idx_v[pl.ds(j * 16, 16)]
        rid_v[pl.ds(j * 16, 16)] = b * (N_TOT // 32) + (ix >> 5)
        off_v[pl.ds(j * 16, 16)] = (ix & 31) * 4
        pos = obase + j * 16 + lanes
        m0 = ix < N_LIDAR
        m2 = ix >= N_LIDAR + N_PROP
        m1 = jnp.logical_and(jnp.logical_not(m0), jnp.logical_not(m2))
        plsc.store_compressed(r0_v.at[pl.ds(p0, 16)], b * N_LIDAR + ix, mask=m0)
        plsc.store_compressed(q0_v.at[pl.ds(p0, 16)], pos, mask=m0)
        plsc.store_compressed(r1_v.at[pl.ds(p1, 16)], b * N_PROP + (ix - N_LIDAR),
                              mask=m1)
        plsc.store_compressed(q1_v.at[pl.ds(p1, 16)], pos, mask=m1)
        plsc.store_compressed(r2_v.at[pl.ds(p2, 16)],
                              b * N_GLOB + (ix - (N_LIDAR + N_PROP)), mask=m2)
        plsc.store_compressed(q2_v.at[pl.ds(p2, 16)], pos, mask=m2)
        p0 = p0 + jnp.sum(m0.astype(jnp.int32))
        p1 = p1 + jnp.sum(m1.astype(jnp.int32))
        p2 = p2 + jnp.sum(m2.astype(jnp.int32))
        return p0, p1, p2

    lax.fori_loop(0, ROWS_W // 16, _split,
                  (jnp.int32(0), jnp.int32(0), jnp.int32(0)))

    for s3, (tab, rv, qv) in enumerate(
            ((lq_hbm, r0_v, q0_v), (pq_hbm, r1_v, q1_v), (gq_hbm, r2_v, q2_v))):
        for h in range(2):
            for cc in range(HALF // 16):
                rows2_v[h, pl.ds(cc * 16, 16)] = rv[pl.ds(h * HALF + cc * 16, 16)]
                pos2_v[h, pl.ds(cc * 16, 16)] = qv[pl.ds(h * HALF + cc * 16, 16)]
        for h in range(2):
            pltpu.async_copy(tab.at[rows2_v.at[h]], qrow_v, sem).wait()

            def _embadd(r, _, s3=s3):
                for c8 in range(D // 16):
                    e = emb_v[pl.ds(s3 * D + c8 * 16, 16)]
                    qrow_v[r, pl.ds(c8 * 16, 16)] = qrow_v[r, pl.ds(c8 * 16, 16)] + e
                return 0

            lax.fori_loop(0, HALF, _embadd, 0)
            pltpu.async_copy(qrow_v, outq_hbm.at[pos2_v.at[h]], sem).wait()

    for h in range(2):
        for cc in range(HALF // 16):
            rid2_v[h, pl.ds(cc * 16, 16)] = rid_v[pl.ds(h * HALF + cc * 16, 16)]
    for h in range(2):
        pltpu.async_copy(refs128_hbm.at[rid2_v.at[h]], rbig_v, sem).wait()

        def _rext(j, _, h=h):
            rloc = j * 16 + lanes
            off = off_v[pl.ds(h * HALF + j * 16, 16)]
            for ccc in range(4):
                vals = plsc.load_gather(rbig_v, [rloc, off + ccc])
                plsc.store_scatter(rrow_v, [rloc * 4 + ccc], vals)
            return 0

        lax.fori_loop(0, HALF // 16, _rext, 0)
        pltpu.sync_copy(rrow_v, outr4_hbm.at[pl.ds((obase + h * HALF) * 4,
                                                   HALF * 4)])


@jax.jit
def _gather_call(top_idx, lq, pq, gq, refs128, embf):
    mesh = plsc.VectorSubcoreMesh(core_axis_name="c", subcore_axis_name="s")
    f = pl.kernel(
        _gather_body,
        out_type=[
            jax.ShapeDtypeStruct((B * PADK, D), jnp.float32),
            jax.ShapeDtypeStruct((B * PADK * 4,), jnp.float32),
        ],
        mesh=mesh,
        compiler_params=pltpu.CompilerParams(needs_layout_passes=False),
        scratch_types=[
            pltpu.VMEM((ROWS_W,), jnp.int32),
            pltpu.VMEM((ROWS_W,), jnp.int32),
            pltpu.VMEM((ROWS_W,), jnp.int32),
            pltpu.VMEM((2, HALF), jnp.int32),
            pltpu.VMEM((2, HALF), jnp.int32),
            pltpu.VMEM((2, HALF), jnp.int32),
            pltpu.VMEM((ROWS_W + 16,), jnp.int32),
            pltpu.VMEM((ROWS_W + 16,), jnp.int32),
            pltpu.VMEM((ROWS_W + 16,), jnp.int32),
            pltpu.VMEM((ROWS_W + 16,), jnp.int32),
            pltpu.VMEM((ROWS_W + 16,), jnp.int32),
            pltpu.VMEM((ROWS_W + 16,), jnp.int32),
            pltpu.VMEM((HALF * 4,), jnp.float32),
            pltpu.VMEM((HALF, D), jnp.float32),
            pltpu.VMEM((HALF, D), jnp.float32),
            pltpu.VMEM((3 * D,), jnp.float32),
            pltpu.SemaphoreType.DMA,
        ],
    )
    return f(top_idx, lq, pq, gq, refs128, embf)


def kernel(lidar_queries, lidar_refs, lidar_scores,
           proposal_queries, proposal_refs, proposal_scores,
           global_queries, global_refs, global_scores,
           source_embeddings, W1, b1, W2, b2):
    ks, kl, asc = _score_call(lidar_queries, lidar_scores,
                              proposal_queries, proposal_scores,
                              global_queries, global_scores,
                              source_embeddings, W1, b1, W2, b2)
    top_idx, outs, outl, outsrc = _topk_call(ks, asc, kl)
    refs128 = jnp.pad(
        jnp.concatenate([lidar_refs, proposal_refs, global_refs], axis=1),
        ((0, 0), (0, 0), (0, 1))).reshape(B * N_TOT // 32, 128)
    outq, outr4 = _gather_call(
        top_idx,
        lidar_queries.reshape(B * N_LIDAR, D),
        proposal_queries.reshape(B * N_PROP, D),
        global_queries.reshape(B * N_GLOB, D),
        refs128, source_embeddings.reshape(3 * D))
    gathered_queries = outq.reshape(B, PADK, D)[:, :KEEP]
    gathered_refs = outr4.reshape(B, PADK, 4)[:, :KEEP, :3]
    gathered_scores = outs[:, :KEEP]
    gathered_sources = outsrc[:, :KEEP]
    gathered_logits = outl[:, :KEEP]
    return (gathered_queries, gathered_refs, gathered_scores, gathered_sources, gathered_logits)


# speedup vs baseline: 1.3250x; 1.3250x over previous
import functools

import jax
import jax.numpy as jnp
from jax import lax
from jax.experimental import pallas as pl
from jax.experimental.pallas import tpu as pltpu
from jax.experimental.pallas import tpu_sc as plsc

B = 8
N_LIDAR, N_PROP, N_GLOB = 16384, 8192, 8192
N_TOT = N_LIDAR + N_PROP + N_GLOB
D = 128
KEEP = 1000
CHUNK = 2048
N_CH_L = N_LIDAR // CHUNK
N_CH_P = N_PROP // CHUNK
N_CH_G = N_GLOB // CHUNK
N_CH = N_CH_L + N_CH_P + N_CH_G


def _score_body(emb_ref, w1_ref, b1_ref, w2_ref, b2_ref,
                lq_ref, ls_ref, pq_ref, ps_ref, gq_ref, gs_ref,
                ks_ref, kl_ref, as_ref):
    g = pl.program_id(1)
    is_l = g < N_CH_L
    is_p = jnp.logical_and(g >= N_CH_L, g < N_CH_L + N_CH_P)
    q = jnp.where(is_l, lq_ref[0, 0], jnp.where(is_p, pq_ref[0, 0], gq_ref[0, 0]))
    s = jnp.where(is_l, ls_ref[0, 0, 0], jnp.where(is_p, ps_ref[0, 0, 0], gs_ref[0, 0, 0]))
    e = jnp.where(is_l, emb_ref[0:1, :], jnp.where(is_p, emb_ref[1:2, :], emb_ref[2:3, :]))
    aq = q + e
    feat = jnp.concatenate([aq, s[:, None]], axis=1)
    h = jnp.maximum(jnp.dot(feat, w1_ref[...], preferred_element_type=jnp.float32)
                    + b1_ref[0:1, :], 0.0)
    logits = lax.dot_general(
        w2_ref[...], h, dimension_numbers=(((0,), (1,)), ((), ())),
        preferred_element_type=jnp.float32)[0] + b2_ref[0, 0]
    ks_ref[0, 0, 0, :] = logits + s
    kl_ref[0, 0, 0, :] = logits
    as_ref[0, 0, 0, :] = s


@functools.partial(jax.jit, static_argnames=("interpret",))
def _score_call(lq, ls, pq, ps, gq, gs, emb, w1, b1, w2, b2, interpret=False):
    ls3 = ls.reshape(B, N_CH_L, 1, CHUNK)
    ps3 = ps.reshape(B, N_CH_P, 1, CHUNK)
    gs3 = gs.reshape(B, N_CH_G, 1, CHUNK)
    grid = (B, N_CH)

    def qmap(lo, hi):
        return lambda b, g: (b, jnp.clip(g - lo, 0, hi - lo - 1), 0, 0)

    def smap(lo, hi):
        return lambda b, g: (b, jnp.clip(g - lo, 0, hi - lo - 1), 0, 0)

    out = pl.pallas_call(
        _score_body,
        grid=grid,
        in_specs=[
            pl.BlockSpec((3, D), lambda b, g: (0, 0)),
            pl.BlockSpec((D + 1, D), lambda b, g: (0, 0)),
            pl.BlockSpec((1, D), lambda b, g: (0, 0)),
            pl.BlockSpec((D, 1), lambda b, g: (0, 0)),
            pl.BlockSpec((1, 1), lambda b, g: (0, 0)),
            pl.BlockSpec((1, 1, CHUNK, D), qmap(0, N_CH_L)),
            pl.BlockSpec((1, 1, 1, CHUNK), smap(0, N_CH_L)),
            pl.BlockSpec((1, 1, CHUNK, D), qmap(N_CH_L, N_CH_L + N_CH_P)),
            pl.BlockSpec((1, 1, 1, CHUNK), smap(N_CH_L, N_CH_L + N_CH_P)),
            pl.BlockSpec((1, 1, CHUNK, D), qmap(N_CH_L + N_CH_P, N_CH)),
            pl.BlockSpec((1, 1, 1, CHUNK), smap(N_CH_L + N_CH_P, N_CH)),
        ],
        out_specs=[
            pl.BlockSpec((1, 1, 1, CHUNK), lambda b, g: (b, g, 0, 0)),
            pl.BlockSpec((1, 1, 1, CHUNK), lambda b, g: (b, g, 0, 0)),
            pl.BlockSpec((1, 1, 1, CHUNK), lambda b, g: (b, g, 0, 0)),
        ],
        out_shape=[jax.ShapeDtypeStruct((B, N_CH, 1, CHUNK), jnp.float32)] * 3,
        interpret=interpret,
    )(emb, w1, b1.reshape(1, D), w2, b2.reshape(1, 1),
      lq.reshape(B, N_CH_L, CHUNK, D), ls3,
      pq.reshape(B, N_CH_P, CHUNK, D), ps3,
      gq.reshape(B, N_CH_G, CHUNK, D), gs3)
    ks, kl, asc = (o.reshape(B, N_TOT) for o in out)
    return ks, kl, asc


N_VREG = N_TOT // 16
PADK = 1024


def _u32(x):
    return x.astype(jnp.uint32)


def _vperm(x, perm):
    return lax.gather(
        x, perm[:, None],
        lax.GatherDimensionNumbers(offset_dims=(), collapsed_slice_dims=(0,),
                                   start_index_map=(0,)),
        (1,), mode=lax.GatherScatterMode.PROMISE_IN_BOUNDS)


def _topk_body(ks_hbm, asc_hbm, kl_hbm, out_hbm, outs_hbm, outl_hbm,
               outsrc_hbm, key_v, asc_v, kl_v, hist_v, gtk_v, gti_v, eqi_v,
               srtk_v, srti_v, gs_v, gl_v, gsrc_v, sem):
    c = lax.axis_index("c")
    s = lax.axis_index("s")
    wid = s * 2 + c

    @pl.when(wid < B)
    def _run():
        b = wid
        lanes = lax.iota(jnp.int32, 16)
        pltpu.sync_copy(ks_hbm.at[b], key_v)
        pltpu.sync_copy(asc_hbm.at[b], asc_v)
        pltpu.sync_copy(kl_hbm.at[b], kl_v)

        def _clear_hist(i, _):
            hist_v[pl.ds(i * 16, 16)] = jnp.zeros((16,), jnp.int32)
            return 0

        ones = jnp.ones((16,), jnp.int32)

        lax.fori_loop(0, 256, _clear_hist, 0)

        def _p1(i, _):
            x = key_v[pl.ds(i * 16, 16)]
            u = x.astype(jnp.uint32)
            neg = u >> 31
            m = (jnp.uint32(0) - neg) | jnp.uint32(0x80000000)
            k = u ^ m
            key_v[pl.ds(i * 16, 16)] = k.astype(jnp.int32)
            d = (k >> 24).astype(jnp.int32)
            plsc.addupdate_scatter(hist_v, [d * 16 + lanes], ones)
            return 0

        lax.fori_loop(0, N_VREG, _p1, 0)

        def _scan_bins(krem):
            def bscan(i, carry):
                sel, above, cum, found = carry
                bin_ = 255 - i
                cnt = jnp.sum(hist_v[pl.ds(bin_ * 16, 16)])
                newcum = cum + cnt
                hit = jnp.logical_and(jnp.logical_not(found), newcum >= krem)
                sel = jnp.where(hit, bin_, sel)
                above = jnp.where(hit, cum, above)
                return sel, above, newcum, jnp.logical_or(found, hit)

            sel, above, _, _ = lax.fori_loop(
                0, 256, bscan, (jnp.int32(0), jnp.int32(0), jnp.int32(0),
                                jnp.bool_(False)))
            return sel, above

        krem = jnp.int32(KEEP)
        sel, above = _scan_bins(krem)
        prefix = _u32(sel)
        krem = krem - above

        for shift in (16, 8, 0):
            lax.fori_loop(0, 256, _clear_hist, 0)
            pfx = prefix

            def _pp(i, _, shift=shift, pfx=pfx):
                k = key_v[pl.ds(i * 16, 16)].astype(jnp.uint32)
                msk = (k >> (shift + 8)) == pfx
                d = ((k >> shift) & jnp.uint32(0xFF)).astype(jnp.int32)
                plsc.addupdate_scatter(hist_v, [d * 16 + lanes], ones, mask=msk)
                return 0

            lax.fori_loop(0, N_VREG, _pp, 0)
            sel, above = _scan_bins(krem)
            prefix = (prefix << 8) | _u32(sel)
            krem = krem - above

        t = prefix

        def _comp(i, carry):
            pg, pe = carry
            k = key_v[pl.ds(i * 16, 16)].astype(jnp.uint32)
            idxv = i * 16 + lanes
            m_gt = k > t
            m_eq = k == t
            plsc.store_compressed(gtk_v.at[pl.ds(pg, 16)], k.astype(jnp.int32), mask=m_gt)
            plsc.store_compressed(gti_v.at[pl.ds(pg, 16)], idxv, mask=m_gt)

            @pl.when(pe < PADK)
            def _():
                plsc.store_compressed(eqi_v.at[pl.ds(pe, 16)], idxv, mask=m_eq)

            pg = pg + jnp.sum(m_gt.astype(jnp.int32))
            pe = pe + jnp.sum(m_eq.astype(jnp.int32))
            return pg, pe

        n_gt, _ = lax.fori_loop(0, N_VREG, _comp, (jnp.int32(0), jnp.int32(0)))

        def _init(i, _):
            srtk_v[pl.ds(i * 16, 16)] = jnp.zeros((16,), jnp.uint32)
            srti_v[pl.ds(i * 16, 16)] = jnp.full((16,), 0x7FFFFFFF, jnp.int32)
            return 0

        lax.fori_loop(0, PADK // 16, _init, 0)

        def _cgt(i, _):
            pos = i * 16 + lanes
            m = pos < n_gt
            kk = gtk_v[pl.ds(i * 16, 16)].astype(jnp.uint32)
            ii = gti_v[pl.ds(i * 16, 16)]
            ok = srtk_v[pl.ds(i * 16, 16)]
            oi = srti_v[pl.ds(i * 16, 16)]
            srtk_v[pl.ds(i * 16, 16)] = jnp.where(m, kk, ok)
            srti_v[pl.ds(i * 16, 16)] = jnp.where(m, ii, oi)
            return 0

        lax.fori_loop(0, PADK // 16, _cgt, 0)

        def _ceq(j, _):
            jpos = j * 16 + lanes
            m = (n_gt + jpos) < KEEP
            e = eqi_v[pl.ds(j * 16, 16)]
            base = n_gt + j * 16
            ok = srtk_v[pl.ds(base, 16)]
            oi = srti_v[pl.ds(base, 16)]
            srtk_v[pl.ds(base, 16)] = jnp.where(m, jnp.full((16,), 1, jnp.uint32) * t, ok)
            srti_v[pl.ds(base, 16)] = jnp.where(m, e, oi)
            return 0

        lax.fori_loop(0, (KEEP + 15) // 16, _ceq, 0)

        perm_base = lanes

        def _lex_ge(ka, ia, kb, ib):
            return jnp.logical_or(
                ka > kb, jnp.logical_and(ka == kb, ia < ib))

        for size in (2, 4, 8, 16, 32, 64, 128, 256, 512, 1024):
            stride = size // 2
            while stride >= 16:
                w = stride // 16

                def _pair(p, _, w=w, size=size):
                    va = ((p & ~(w - 1)) << 1) | (p & (w - 1))
                    vb = va + w
                    dsc = ((va * 16) & size) == 0
                    ka = srtk_v[pl.ds(va * 16, 16)]
                    ia = srti_v[pl.ds(va * 16, 16)]
                    kb = srtk_v[pl.ds(vb * 16, 16)]
                    ib = srti_v[pl.ds(vb * 16, 16)]
                    ge = _lex_ge(ka, ia, kb, ib)
                    m = jnp.where(dsc, ge, jnp.logical_not(ge))
                    srtk_v[pl.ds(va * 16, 16)] = jnp.where(m, ka, kb)
                    srti_v[pl.ds(va * 16, 16)] = jnp.where(m, ia, ib)
                    srtk_v[pl.ds(vb * 16, 16)] = jnp.where(m, kb, ka)
                    srti_v[pl.ds(vb * 16, 16)] = jnp.where(m, ib, ia)
                    return 0

                lax.fori_loop(0, PADK // 32, _pair, 0)
                stride //= 2
            while stride >= 1:
                perm = perm_base ^ stride

                def _intra(v, _, stride=stride, size=size, perm=perm):
                    kk = srtk_v[pl.ds(v * 16, 16)]
                    ii = srti_v[pl.ds(v * 16, 16)]
                    kp = _vperm(kk, perm)
                    ip = _vperm(ii, perm)
                    low = (lanes & stride) == 0
                    dsc = ((v * 16 + lanes) & size) == 0
                    ge = _lex_ge(kk, ii, kp, ip)
                    cond = ge == (low == dsc)
                    srtk_v[pl.ds(v * 16, 16)] = jnp.where(cond, kk, kp)
                    srti_v[pl.ds(v * 16, 16)] = jnp.where(cond, ii, ip)
                    return 0

                lax.fori_loop(0, PADK // 16, _intra, 0)
                stride //= 2

        srti_v[pl.ds(KEEP, 16)] = lanes * 8
        srti_v[pl.ds(PADK - 16, 16)] = (lanes + 16) * 8

        def _gout(i, _):
            sidx = srti_v[pl.ds(i * 16, 16)]
            gs_v[pl.ds(i * 16, 16)] = plsc.load_gather(asc_v, [sidx])
            gl_v[pl.ds(i * 16, 16)] = plsc.load_gather(kl_v, [sidx])
            gsrc_v[pl.ds(i * 16, 16)] = (
                (sidx >= N_LIDAR).astype(jnp.int32)
                + (sidx >= N_LIDAR + N_PROP).astype(jnp.int32))
            return 0

        lax.fori_loop(0, PADK // 16, _gout, 0)
        pltpu.sync_copy(srti_v, out_hbm.at[b])
        pltpu.sync_copy(gs_v, outs_hbm.at[b])
        pltpu.sync_copy(gl_v, outl_hbm.at[b])
        pltpu.sync_copy(gsrc_v, outsrc_hbm.at[b])


@jax.jit
def _topk_call(ks, asc, kl):
    ksb = lax.bitcast_convert_type(ks, jnp.int32)
    mesh = plsc.VectorSubcoreMesh(core_axis_name="c", subcore_axis_name="s")
    f = pl.kernel(
        _topk_body,
        out_type=[
            jax.ShapeDtypeStruct((B, PADK), jnp.int32),
            jax.ShapeDtypeStruct((B, PADK), jnp.float32),
            jax.ShapeDtypeStruct((B, PADK), jnp.float32),
            jax.ShapeDtypeStruct((B, PADK), jnp.int32),
        ],
        mesh=mesh,
        compiler_params=pltpu.CompilerParams(needs_layout_passes=False),
        scratch_types=[
            pltpu.VMEM((N_TOT,), jnp.int32),
            pltpu.VMEM((N_TOT,), jnp.float32),
            pltpu.VMEM((N_TOT,), jnp.float32),
            pltpu.VMEM((256 * 16,), jnp.int32),
            pltpu.VMEM((PADK + 16,), jnp.int32),
            pltpu.VMEM((PADK + 16,), jnp.int32),
            pltpu.VMEM((PADK + 16,), jnp.int32),
            pltpu.VMEM((PADK,), jnp.uint32),
            pltpu.VMEM((PADK,), jnp.int32),
            pltpu.VMEM((PADK,), jnp.float32),
            pltpu.VMEM((PADK,), jnp.float32),
            pltpu.VMEM((PADK,), jnp.int32),
            pltpu.SemaphoreType.DMA,
        ],
    )
    return f(ksb, asc, kl)


ROWS_W = PADK // 4
HALF = 128


def _gather_body(top_hbm, lq_hbm, pq_hbm, gq_hbm, refs128_hbm, emb_hbm,
                 outq_hbm, outr4_hbm,
                 idx_v, rid_v, off_v, rid2_v, rows2_v, pos2_v,
                 r0_v, r1_v, r2_v, q0_v, q1_v, q2_v,
                 rrow_v, rbig_v, qrow_v, emb_v, sem):
    c = lax.axis_index("c")
    s = lax.axis_index("s")
    wid = s * 2 + c
    b = wid // 4
    part = wid % 4
    lanes = lax.iota(jnp.int32, 16)
    obase = b * PADK + part * ROWS_W

    pltpu.sync_copy(top_hbm.at[b, pl.ds(part * ROWS_W, ROWS_W)], idx_v)
    pltpu.sync_copy(emb_hbm, emb_v)

    def _dflt(j, _):
        safe = b * N_PROP + j * 16 + lanes
        dump = jnp.full((16,), b * PADK + PADK - 1, jnp.int32)
        r0_v[pl.ds(j * 16, 16)] = safe
        r1_v[pl.ds(j * 16, 16)] = safe
        r2_v[pl.ds(j * 16, 16)] = safe
        q0_v[pl.ds(j * 16, 16)] = dump
        q1_v[pl.ds(j * 16, 16)] = dump
        q2_v[pl.ds(j * 16, 16)] = dump
        return 0

    lax.fori_loop(0, ROWS_W // 16 + 1, _dflt, 0)

    def _split(j, carry):
        p0, p1, p2 = carry
        ix = idx_v[pl.ds(j * 16, 16)]
        rid_v[pl.ds(j * 16, 16)] = b * (N_TOT // 32) + (ix >> 5)
        off_v[pl.ds(j * 16, 16)] = (ix & 31) * 4
        pos = obase + j * 16 + lanes
        m0 = ix < N_LIDAR
        m2 = ix >= N_LIDAR + N_PROP
        m1 = jnp.logical_and(jnp.logical_not(m0), jnp.logical_not(m2))
        plsc.store_compressed(r0_v.at[pl.ds(p0, 16)], b * N_LIDAR + ix, mask=m0)
        plsc.store_compressed(q0_v.at[pl.ds(p0, 16)], pos, mask=m0)
        plsc.store_compressed(r1_v.at[pl.ds(p1, 16)], b * N_PROP + (ix - N_LIDAR),
                              mask=m1)
        plsc.store_compressed(q1_v.at[pl.ds(p1, 16)], pos, mask=m1)
        plsc.store_compressed(r2_v.at[pl.ds(p2, 16)],
                              b * N_GLOB + (ix - (N_LIDAR + N_PROP)), mask=m2)
        plsc.store_compressed(q2_v.at[pl.ds(p2, 16)], pos, mask=m2)
        p0 = p0 + jnp.sum(m0.astype(jnp.int32))
        p1 = p1 + jnp.sum(m1.astype(jnp.int32))
        p2 = p2 + jnp.sum(m2.astype(jnp.int32))
        return p0, p1, p2

    lax.fori_loop(0, ROWS_W // 16, _split,
                  (jnp.int32(0), jnp.int32(0), jnp.int32(0)))

    for s3, (tab, rv, qv) in enumerate(
            ((lq_hbm, r0_v, q0_v), (pq_hbm, r1_v, q1_v), (gq_hbm, r2_v, q2_v))):
        for h in range(2):
            for cc in range(HALF // 16):
                rows2_v[h, pl.ds(cc * 16, 16)] = rv[pl.ds(h * HALF + cc * 16, 16)]
                pos2_v[h, pl.ds(cc * 16, 16)] = qv[pl.ds(h * HALF + cc * 16, 16)]
        for h in range(2):
            pltpu.async_copy(tab.at[rows2_v.at[h]], qrow_v, sem).wait()

            def _embadd(r, _, s3=s3):
                for c8 in range(D // 16):
                    e = emb_v[pl.ds(s3 * D + c8 * 16, 16)]
                    qrow_v[r, pl.ds(c8 * 16, 16)] = qrow_v[r, pl.ds(c8 * 16, 16)] + e
                return 0

            lax.fori_loop(0, HALF, _embadd, 0)
            pltpu.async_copy(qrow_v, outq_hbm.at[pos2_v.at[h]], sem).wait()

    for h in range(2):
        for cc in range(HALF // 16):
            rid2_v[h, pl.ds(cc * 16, 16)] = rid_v[pl.ds(h * HALF + cc * 16, 16)]
    for h in range(2):
        pltpu.async_copy(refs128_hbm.at[rid2_v.at[h]], rbig_v, sem).wait()

        def _rext(j, _, h=h):
            rloc = j * 16 + lanes
            off = off_v[pl.ds(h * HALF + j * 16, 16)]
            for ccc in range(4):
                vals = plsc.load_gather(rbig_v, [rloc, off + ccc])
                plsc.store_scatter(rrow_v, [rloc * 4 + ccc], vals)
            return 0

        lax.fori_loop(0, HALF // 16, _rext, 0)
        pltpu.sync_copy(rrow_v, outr4_hbm.at[pl.ds((obase + h * HALF) * 4,
                                                   HALF * 4)])


@jax.jit
def _gather_call(top_idx, lq, pq, gq, refs128, embf):
    mesh = plsc.VectorSubcoreMesh(core_axis_name="c", subcore_axis_name="s")
    f = pl.kernel(
        _gather_body,
        out_type=[
            jax.ShapeDtypeStruct((B * PADK, D), jnp.float32),
            jax.ShapeDtypeStruct((B * PADK * 4,), jnp.float32),
        ],
        mesh=mesh,
        compiler_params=pltpu.CompilerParams(needs_layout_passes=False),
        scratch_types=[
            pltpu.VMEM((ROWS_W,), jnp.int32),
            pltpu.VMEM((ROWS_W,), jnp.int32),
            pltpu.VMEM((ROWS_W,), jnp.int32),
            pltpu.VMEM((2, HALF), jnp.int32),
            pltpu.VMEM((2, HALF), jnp.int32),
            pltpu.VMEM((2, HALF), jnp.int32),
            pltpu.VMEM((ROWS_W + 16,), jnp.int32),
            pltpu.VMEM((ROWS_W + 16,), jnp.int32),
            pltpu.VMEM((ROWS_W + 16,), jnp.int32),
            pltpu.VMEM((ROWS_W + 16,), jnp.int32),
            pltpu.VMEM((ROWS_W + 16,), jnp.int32),
            pltpu.VMEM((ROWS_W + 16,), jnp.int32),
            pltpu.VMEM((HALF * 4,), jnp.float32),
            pltpu.VMEM((HALF, D), jnp.float32),
            pltpu.VMEM((HALF, D), jnp.float32),
            pltpu.VMEM((3 * D,), jnp.float32),
            pltpu.SemaphoreType.DMA,
        ],
    )
    return f(top_idx, lq, pq, gq, refs128, embf)


def kernel(lidar_queries, lidar_refs, lidar_scores,
           proposal_queries, proposal_refs, proposal_scores,
           global_queries, global_refs, global_scores,
           source_embeddings, W1, b1, W2, b2):
    ks, kl, asc = _score_call(lidar_queries, lidar_scores,
                              proposal_queries, proposal_scores,
                              global_queries, global_scores,
                              source_embeddings, W1, b1, W2, b2)
    top_idx, outs, outl, outsrc = _topk_call(ks, asc, kl)
    refs128 = jnp.pad(
        jnp.concatenate([lidar_refs, proposal_refs, global_refs], axis=1),
        ((0, 0), (0, 0), (0, 1))).reshape(B * N_TOT // 32, 128)
    outq, outr4 = _gather_call(
        top_idx,
        lidar_queries.reshape(B * N_LIDAR, D),
        proposal_queries.reshape(B * N_PROP, D),
        global_queries.reshape(B * N_GLOB, D),
        refs128, source_embeddings.reshape(3 * D))
    gathered_queries = outq.reshape(B, PADK, D)[:, :KEEP]
    gathered_refs = outr4.reshape(B, PADK, 4)[:, :KEEP, :3]
    gathered_scores = outs[:, :KEEP]
    gathered_sources = outsrc[:, :KEEP]
    gathered_logits = outl[:, :KEEP]
    return (gathered_queries, gathered_refs, gathered_scores, gathered_sources, gathered_logits)


# speedup vs baseline: 1.4771x; 1.1148x over previous
import functools

import jax
import jax.numpy as jnp
from jax import lax
from jax.experimental import pallas as pl
from jax.experimental.pallas import tpu as pltpu
from jax.experimental.pallas import tpu_sc as plsc

B = 8
N_LIDAR, N_PROP, N_GLOB = 16384, 8192, 8192
N_TOT = N_LIDAR + N_PROP + N_GLOB
D = 128
KEEP = 1000
CHUNK = 2048
N_CH_L = N_LIDAR // CHUNK
N_CH_P = N_PROP // CHUNK
N_CH_G = N_GLOB // CHUNK
N_CH = N_CH_L + N_CH_P + N_CH_G


def _score_body(emb_ref, w1_ref, b1_ref, w2_ref, b2_ref,
                lq_ref, ls_ref, pq_ref, ps_ref, gq_ref, gs_ref,
                ks_ref, kl_ref, as_ref):
    g = pl.program_id(1)
    is_l = g < N_CH_L
    is_p = jnp.logical_and(g >= N_CH_L, g < N_CH_L + N_CH_P)
    q = jnp.where(is_l, lq_ref[0, 0], jnp.where(is_p, pq_ref[0, 0], gq_ref[0, 0]))
    s = jnp.where(is_l, ls_ref[0, 0, 0], jnp.where(is_p, ps_ref[0, 0, 0], gs_ref[0, 0, 0]))
    e = jnp.where(is_l, emb_ref[0:1, :], jnp.where(is_p, emb_ref[1:2, :], emb_ref[2:3, :]))
    aq = q + e
    feat = jnp.concatenate([aq, s[:, None]], axis=1)
    h = jnp.maximum(jnp.dot(feat, w1_ref[...], preferred_element_type=jnp.float32)
                    + b1_ref[0:1, :], 0.0)
    logits = lax.dot_general(
        w2_ref[...], h, dimension_numbers=(((0,), (1,)), ((), ())),
        preferred_element_type=jnp.float32)[0] + b2_ref[0, 0]
    ks_ref[0, 0, 0, :] = logits + s
    kl_ref[0, 0, 0, :] = logits
    as_ref[0, 0, 0, :] = s


@functools.partial(jax.jit, static_argnames=("interpret",))
def _score_call(lq, ls, pq, ps, gq, gs, emb, w1, b1, w2, b2, interpret=False):
    ls3 = ls.reshape(B, N_CH_L, 1, CHUNK)
    ps3 = ps.reshape(B, N_CH_P, 1, CHUNK)
    gs3 = gs.reshape(B, N_CH_G, 1, CHUNK)
    grid = (B, N_CH)

    def qmap(lo, hi):
        return lambda b, g: (b, jnp.clip(g - lo, 0, hi - lo - 1), 0, 0)

    def smap(lo, hi):
        return lambda b, g: (b, jnp.clip(g - lo, 0, hi - lo - 1), 0, 0)

    out = pl.pallas_call(
        _score_body,
        grid=grid,
        in_specs=[
            pl.BlockSpec((3, D), lambda b, g: (0, 0)),
            pl.BlockSpec((D + 1, D), lambda b, g: (0, 0)),
            pl.BlockSpec((1, D), lambda b, g: (0, 0)),
            pl.BlockSpec((D, 1), lambda b, g: (0, 0)),
            pl.BlockSpec((1, 1), lambda b, g: (0, 0)),
            pl.BlockSpec((1, 1, CHUNK, D), qmap(0, N_CH_L)),
            pl.BlockSpec((1, 1, 1, CHUNK), smap(0, N_CH_L)),
            pl.BlockSpec((1, 1, CHUNK, D), qmap(N_CH_L, N_CH_L + N_CH_P)),
            pl.BlockSpec((1, 1, 1, CHUNK), smap(N_CH_L, N_CH_L + N_CH_P)),
            pl.BlockSpec((1, 1, CHUNK, D), qmap(N_CH_L + N_CH_P, N_CH)),
            pl.BlockSpec((1, 1, 1, CHUNK), smap(N_CH_L + N_CH_P, N_CH)),
        ],
        out_specs=[
            pl.BlockSpec((1, 1, 1, CHUNK), lambda b, g: (b, g, 0, 0)),
            pl.BlockSpec((1, 1, 1, CHUNK), lambda b, g: (b, g, 0, 0)),
            pl.BlockSpec((1, 1, 1, CHUNK), lambda b, g: (b, g, 0, 0)),
        ],
        out_shape=[jax.ShapeDtypeStruct((B, N_CH, 1, CHUNK), jnp.float32)] * 3,
        interpret=interpret,
    )(emb, w1, b1.reshape(1, D), w2, b2.reshape(1, 1),
      lq.reshape(B, N_CH_L, CHUNK, D), ls3,
      pq.reshape(B, N_CH_P, CHUNK, D), ps3,
      gq.reshape(B, N_CH_G, CHUNK, D), gs3)
    ks, kl, asc = (o.reshape(B, N_TOT) for o in out)
    return ks, kl, asc


N_VREG = N_TOT // 16
PADK = 1024


def _u32(x):
    return x.astype(jnp.uint32)


def _vperm(x, perm):
    return lax.gather(
        x, perm[:, None],
        lax.GatherDimensionNumbers(offset_dims=(), collapsed_slice_dims=(0,),
                                   start_index_map=(0,)),
        (1,), mode=lax.GatherScatterMode.PROMISE_IN_BOUNDS)


def _topk_body(ks_hbm, asc_hbm, kl_hbm, out_hbm, outs_hbm, outl_hbm,
               outsrc_hbm, key_v, asc_v, kl_v, hist_v, gtk_v, gti_v, eqi_v,
               srtk_v, srti_v, gs_v, gl_v, gsrc_v, sem):
    c = lax.axis_index("c")
    s = lax.axis_index("s")
    wid = s * 2 + c

    @pl.when(wid < B)
    def _run():
        b = wid
        lanes = lax.iota(jnp.int32, 16)
        pltpu.sync_copy(ks_hbm.at[b], key_v)
        pltpu.sync_copy(asc_hbm.at[b], asc_v)
        pltpu.sync_copy(kl_hbm.at[b], kl_v)

        def _clear_hist(i, _):
            hist_v[pl.ds(i * 16, 16)] = jnp.zeros((16,), jnp.int32)
            return 0

        ones = jnp.ones((16,), jnp.int32)

        lax.fori_loop(0, 256, _clear_hist, 0)

        def _p1(i4, _):
            for u4 in range(4):
                i = i4 * 4 + u4
                x = key_v[pl.ds(i * 16, 16)]
                u = x.astype(jnp.uint32)
                neg = u >> 31
                m = (jnp.uint32(0) - neg) | jnp.uint32(0x80000000)
                k = u ^ m
                key_v[pl.ds(i * 16, 16)] = k.astype(jnp.int32)
                d = (k >> 24).astype(jnp.int32)
                plsc.addupdate_scatter(hist_v, [d * 16 + lanes], ones)
            return 0

        lax.fori_loop(0, N_VREG // 4, _p1, 0)

        def _scan_bins(krem):
            def bscan(i, carry):
                sel, above, cum, found = carry
                bin_ = 255 - i
                cnt = jnp.sum(hist_v[pl.ds(bin_ * 16, 16)])
                newcum = cum + cnt
                hit = jnp.logical_and(jnp.logical_not(found), newcum >= krem)
                sel = jnp.where(hit, bin_, sel)
                above = jnp.where(hit, cum, above)
                return sel, above, newcum, jnp.logical_or(found, hit)

            sel, above, _, _ = lax.fori_loop(
                0, 256, bscan, (jnp.int32(0), jnp.int32(0), jnp.int32(0),
                                jnp.bool_(False)))
            return sel, above

        krem = jnp.int32(KEEP)
        sel, above = _scan_bins(krem)
        prefix = _u32(sel)
        krem = krem - above

        for shift in (16, 8, 0):
            lax.fori_loop(0, 256, _clear_hist, 0)
            pfx = prefix

            def _pp(i4, _, shift=shift, pfx=pfx):
                for u4 in range(4):
                    i = i4 * 4 + u4
                    k = key_v[pl.ds(i * 16, 16)].astype(jnp.uint32)
                    msk = (k >> (shift + 8)) == pfx
                    d = ((k >> shift) & jnp.uint32(0xFF)).astype(jnp.int32)
                    plsc.addupdate_scatter(hist_v, [d * 16 + lanes], ones,
                                           mask=msk)
                return 0

            lax.fori_loop(0, N_VREG // 4, _pp, 0)
            sel, above = _scan_bins(krem)
            prefix = (prefix << 8) | _u32(sel)
            krem = krem - above

        t = prefix

        def _comp(i4, carry):
            pg, pe = carry
            for u4 in range(4):
                i = i4 * 4 + u4
                k = key_v[pl.ds(i * 16, 16)].astype(jnp.uint32)
                idxv = i * 16 + lanes
                m_gt = k > t
                m_eq = k == t
                plsc.store_compressed(gtk_v.at[pl.ds(pg, 16)],
                                      k.astype(jnp.int32), mask=m_gt)
                plsc.store_compressed(gti_v.at[pl.ds(pg, 16)], idxv, mask=m_gt)

                @pl.when(pe < PADK)
                def _():
                    plsc.store_compressed(eqi_v.at[pl.ds(pe, 16)], idxv,
                                          mask=m_eq)

                pg = pg + jnp.sum(m_gt.astype(jnp.int32))
                pe = pe + jnp.sum(m_eq.astype(jnp.int32))
            return pg, pe

        n_gt, _ = lax.fori_loop(0, N_VREG // 4, _comp,
                                (jnp.int32(0), jnp.int32(0)))

        def _init(i, _):
            srtk_v[pl.ds(i * 16, 16)] = jnp.zeros((16,), jnp.uint32)
            srti_v[pl.ds(i * 16, 16)] = jnp.full((16,), 0x7FFFFFFF, jnp.int32)
            return 0

        lax.fori_loop(0, PADK // 16, _init, 0)

        def _cgt(i, _):
            pos = i * 16 + lanes
            m = pos < n_gt
            kk = gtk_v[pl.ds(i * 16, 16)].astype(jnp.uint32)
            ii = gti_v[pl.ds(i * 16, 16)]
            ok = srtk_v[pl.ds(i * 16, 16)]
            oi = srti_v[pl.ds(i * 16, 16)]
            srtk_v[pl.ds(i * 16, 16)] = jnp.where(m, kk, ok)
            srti_v[pl.ds(i * 16, 16)] = jnp.where(m, ii, oi)
            return 0

        lax.fori_loop(0, PADK // 16, _cgt, 0)

        def _ceq(j, _):
            jpos = j * 16 + lanes
            m = (n_gt + jpos) < KEEP
            e = eqi_v[pl.ds(j * 16, 16)]
            base = n_gt + j * 16
            ok = srtk_v[pl.ds(base, 16)]
            oi = srti_v[pl.ds(base, 16)]
            srtk_v[pl.ds(base, 16)] = jnp.where(m, jnp.full((16,), 1, jnp.uint32) * t, ok)
            srti_v[pl.ds(base, 16)] = jnp.where(m, e, oi)
            return 0

        lax.fori_loop(0, (KEEP + 15) // 16, _ceq, 0)

        perm_base = lanes

        def _lex_ge(ka, ia, kb, ib):
            return jnp.logical_or(
                ka > kb, jnp.logical_and(ka == kb, ia < ib))

        for size in (2, 4, 8, 16, 32, 64, 128, 256, 512, 1024):
            stride = size // 2
            while stride >= 16:
                w = stride // 16

                def _pair(p, _, w=w, size=size):
                    va = ((p & ~(w - 1)) << 1) | (p & (w - 1))
                    vb = va + w
                    dsc = ((va * 16) & size) == 0
                    ka = srtk_v[pl.ds(va * 16, 16)]
                    ia = srti_v[pl.ds(va * 16, 16)]
                    kb = srtk_v[pl.ds(vb * 16, 16)]
                    ib = srti_v[pl.ds(vb * 16, 16)]
                    ge = _lex_ge(ka, ia, kb, ib)
                    m = jnp.where(dsc, ge, jnp.logical_not(ge))
                    srtk_v[pl.ds(va * 16, 16)] = jnp.where(m, ka, kb)
                    srti_v[pl.ds(va * 16, 16)] = jnp.where(m, ia, ib)
                    srtk_v[pl.ds(vb * 16, 16)] = jnp.where(m, kb, ka)
                    srti_v[pl.ds(vb * 16, 16)] = jnp.where(m, ib, ia)
                    return 0

                lax.fori_loop(0, PADK // 32, _pair, 0)
                stride //= 2
            while stride >= 1:
                perm = perm_base ^ stride

                def _intra(v, _, stride=stride, size=size, perm=perm):
                    kk = srtk_v[pl.ds(v * 16, 16)]
                    ii = srti_v[pl.ds(v * 16, 16)]
                    kp = _vperm(kk, perm)
                    ip = _vperm(ii, perm)
                    low = (lanes & stride) == 0
                    dsc = ((v * 16 + lanes) & size) == 0
                    ge = _lex_ge(kk, ii, kp, ip)
                    cond = ge == (low == dsc)
                    srtk_v[pl.ds(v * 16, 16)] = jnp.where(cond, kk, kp)
                    srti_v[pl.ds(v * 16, 16)] = jnp.where(cond, ii, ip)
                    return 0

                lax.fori_loop(0, PADK // 16, _intra, 0)
                stride //= 2

        srti_v[pl.ds(KEEP, 16)] = lanes * 8
        srti_v[pl.ds(PADK - 16, 16)] = (lanes + 16) * 8

        def _gout(i, _):
            sidx = srti_v[pl.ds(i * 16, 16)]
            gs_v[pl.ds(i * 16, 16)] = plsc.load_gather(asc_v, [sidx])
            gl_v[pl.ds(i * 16, 16)] = plsc.load_gather(kl_v, [sidx])
            gsrc_v[pl.ds(i * 16, 16)] = (
                (sidx >= N_LIDAR).astype(jnp.int32)
                + (sidx >= N_LIDAR + N_PROP).astype(jnp.int32))
            return 0

        lax.fori_loop(0, PADK // 16, _gout, 0)
        pltpu.sync_copy(srti_v, out_hbm.at[b])
        pltpu.sync_copy(gs_v, outs_hbm.at[b])
        pltpu.sync_copy(gl_v, outl_hbm.at[b])
        pltpu.sync_copy(gsrc_v, outsrc_hbm.at[b])


@jax.jit
def _topk_call(ks, asc, kl):
    ksb = lax.bitcast_convert_type(ks, jnp.int32)
    mesh = plsc.VectorSubcoreMesh(core_axis_name="c", subcore_axis_name="s")
    f = pl.kernel(
        _topk_body,
        out_type=[
            jax.ShapeDtypeStruct((B, PADK), jnp.int32),
            jax.ShapeDtypeStruct((B, PADK), jnp.float32),
            jax.ShapeDtypeStruct((B, PADK), jnp.float32),
            jax.ShapeDtypeStruct((B, PADK), jnp.int32),
        ],
        mesh=mesh,
        compiler_params=pltpu.CompilerParams(needs_layout_passes=False),
        scratch_types=[
            pltpu.VMEM((N_TOT,), jnp.int32),
            pltpu.VMEM((N_TOT,), jnp.float32),
            pltpu.VMEM((N_TOT,), jnp.float32),
            pltpu.VMEM((256 * 16,), jnp.int32),
            pltpu.VMEM((PADK + 16,), jnp.int32),
            pltpu.VMEM((PADK + 16,), jnp.int32),
            pltpu.VMEM((PADK + 16,), jnp.int32),
            pltpu.VMEM((PADK,), jnp.uint32),
            pltpu.VMEM((PADK,), jnp.int32),
            pltpu.VMEM((PADK,), jnp.float32),
            pltpu.VMEM((PADK,), jnp.float32),
            pltpu.VMEM((PADK,), jnp.int32),
            pltpu.SemaphoreType.DMA,
        ],
    )
    return f(ksb, asc, kl)


ROWS_W = PADK // 4
HALF = 128


def _gather_body(top_hbm, lq_hbm, pq_hbm, gq_hbm, refs128_hbm, emb_hbm,
                 outq_hbm, outr4_hbm,
                 idx_v, rid_v, off_v, rid2_v, rows2_v, pos2_v,
                 r0_v, r1_v, r2_v, q0_v, q1_v, q2_v,
                 rrow_v, rbig_v, qrow_v, emb_v, sem):
    c = lax.axis_index("c")
    s = lax.axis_index("s")
    wid = s * 2 + c
    b = wid // 4
    part = wid % 4
    lanes = lax.iota(jnp.int32, 16)
    obase = b * PADK + part * ROWS_W

    pltpu.sync_copy(top_hbm.at[b, pl.ds(part * ROWS_W, ROWS_W)], idx_v)
    pltpu.sync_copy(emb_hbm, emb_v)

    def _dflt(j, _):
        safe = b * N_PROP + j * 16 + lanes
        dump = jnp.full((16,), b * PADK + PADK - 1, jnp.int32)
        r0_v[pl.ds(j * 16, 16)] = safe
        r1_v[pl.ds(j * 16, 16)] = safe
        r2_v[pl.ds(j * 16, 16)] = safe
        q0_v[pl.ds(j * 16, 16)] = dump
        q1_v[pl.ds(j * 16, 16)] = dump
        q2_v[pl.ds(j * 16, 16)] = dump
        return 0

    lax.fori_loop(0, ROWS_W // 16 + 1, _dflt, 0)

    def _split(j, carry):
        p0, p1, p2 = carry
        ix = idx_v[pl.ds(j * 16, 16)]
        rid_v[pl.ds(j * 16, 16)] = b * (N_TOT // 32) + (ix >> 5)
        off_v[pl.ds(j * 16, 16)] = (ix & 31) * 4
        pos = obase + j * 16 + lanes
        m0 = ix < N_LIDAR
        m2 = ix >= N_LIDAR + N_PROP
        m1 = jnp.logical_and(jnp.logical_not(m0), jnp.logical_not(m2))
        plsc.store_compressed(r0_v.at[pl.ds(p0, 16)], b * N_LIDAR + ix, mask=m0)
        plsc.store_compressed(q0_v.at[pl.ds(p0, 16)], pos, mask=m0)
        plsc.store_compressed(r1_v.at[pl.ds(p1, 16)], b * N_PROP + (ix - N_LIDAR),
                              mask=m1)
        plsc.store_compressed(q1_v.at[pl.ds(p1, 16)], pos, mask=m1)
        plsc.store_compressed(r2_v.at[pl.ds(p2, 16)],
                              b * N_GLOB + (ix - (N_LIDAR + N_PROP)), mask=m2)
        plsc.store_compressed(q2_v.at[pl.ds(p2, 16)], pos, mask=m2)
        p0 = p0 + jnp.sum(m0.astype(jnp.int32))
        p1 = p1 + jnp.sum(m1.astype(jnp.int32))
        p2 = p2 + jnp.sum(m2.astype(jnp.int32))
        return p0, p1, p2

    p0, p1, p2 = lax.fori_loop(0, ROWS_W // 16, _split,
                               (jnp.int32(0), jnp.int32(0), jnp.int32(0)))

    for s3, (tab, rv, qv, cnt) in enumerate(
            ((lq_hbm, r0_v, q0_v, p0), (pq_hbm, r1_v, q1_v, p1),
             (gq_hbm, r2_v, q2_v, p2))):
        for h in range(2):
            for cc in range(HALF // 16):
                rows2_v[h, pl.ds(cc * 16, 16)] = rv[pl.ds(h * HALF + cc * 16, 16)]
                pos2_v[h, pl.ds(cc * 16, 16)] = qv[pl.ds(h * HALF + cc * 16, 16)]
        for h in range(2):

            def _do_half(h=h, s3=s3):
                pltpu.async_copy(tab.at[rows2_v.at[h]], qrow_v, sem).wait()

                def _embadd(r, _, s3=s3):
                    for c8 in range(D // 16):
                        e = emb_v[pl.ds(s3 * D + c8 * 16, 16)]
                        qrow_v[r, pl.ds(c8 * 16, 16)] = (
                            qrow_v[r, pl.ds(c8 * 16, 16)] + e)
                    return 0

                lax.fori_loop(0, HALF, _embadd, 0)
                pltpu.async_copy(qrow_v, outq_hbm.at[pos2_v.at[h]], sem).wait()

            if h == 0:
                _do_half()
            else:
                pl.when(cnt > HALF)(_do_half)

    for h in range(2):
        for cc in range(HALF // 16):
            rid2_v[h, pl.ds(cc * 16, 16)] = rid_v[pl.ds(h * HALF + cc * 16, 16)]
    for h in range(2):
        pltpu.async_copy(refs128_hbm.at[rid2_v.at[h]], rbig_v, sem).wait()

        def _rext(j, _, h=h):
            rloc = j * 16 + lanes
            off = off_v[pl.ds(h * HALF + j * 16, 16)]
            for ccc in range(4):
                vals = plsc.load_gather(rbig_v, [rloc, off + ccc])
                plsc.store_scatter(rrow_v, [rloc * 4 + ccc], vals)
            return 0

        lax.fori_loop(0, HALF // 16, _rext, 0)
        pltpu.sync_copy(rrow_v, outr4_hbm.at[pl.ds((obase + h * HALF) * 4,
                                                   HALF * 4)])


@jax.jit
def _gather_call(top_idx, lq, pq, gq, refs128, embf):
    mesh = plsc.VectorSubcoreMesh(core_axis_name="c", subcore_axis_name="s")
    f = pl.kernel(
        _gather_body,
        out_type=[
            jax.ShapeDtypeStruct((B * PADK, D), jnp.float32),
            jax.ShapeDtypeStruct((B * PADK * 4,), jnp.float32),
        ],
        mesh=mesh,
        compiler_params=pltpu.CompilerParams(needs_layout_passes=False),
        scratch_types=[
            pltpu.VMEM((ROWS_W,), jnp.int32),
            pltpu.VMEM((ROWS_W,), jnp.int32),
            pltpu.VMEM((ROWS_W,), jnp.int32),
            pltpu.VMEM((2, HALF), jnp.int32),
            pltpu.VMEM((2, HALF), jnp.int32),
            pltpu.VMEM((2, HALF), jnp.int32),
            pltpu.VMEM((ROWS_W + 16,), jnp.int32),
            pltpu.VMEM((ROWS_W + 16,), jnp.int32),
            pltpu.VMEM((ROWS_W + 16,), jnp.int32),
            pltpu.VMEM((ROWS_W + 16,), jnp.int32),
            pltpu.VMEM((ROWS_W + 16,), jnp.int32),
            pltpu.VMEM((ROWS_W + 16,), jnp.int32),
            pltpu.VMEM((HALF * 4,), jnp.float32),
            pltpu.VMEM((HALF, D), jnp.float32),
            pltpu.VMEM((HALF, D), jnp.float32),
            pltpu.VMEM((3 * D,), jnp.float32),
            pltpu.SemaphoreType.DMA,
        ],
    )
    return f(top_idx, lq, pq, gq, refs128, embf)


def kernel(lidar_queries, lidar_refs, lidar_scores,
           proposal_queries, proposal_refs, proposal_scores,
           global_queries, global_refs, global_scores,
           source_embeddings, W1, b1, W2, b2):
    ks, kl, asc = _score_call(lidar_queries, lidar_scores,
                              proposal_queries, proposal_scores,
                              global_queries, global_scores,
                              source_embeddings, W1, b1, W2, b2)
    top_idx, outs, outl, outsrc = _topk_call(ks, asc, kl)
    refs128 = jnp.pad(
        jnp.concatenate([lidar_refs, proposal_refs, global_refs], axis=1),
        ((0, 0), (0, 0), (0, 1))).reshape(B * N_TOT // 32, 128)
    outq, outr4 = _gather_call(
        top_idx,
        lidar_queries.reshape(B * N_LIDAR, D),
        proposal_queries.reshape(B * N_PROP, D),
        global_queries.reshape(B * N_GLOB, D),
        refs128, source_embeddings.reshape(3 * D))
    gathered_queries = outq.reshape(B, PADK, D)[:, :KEEP]
    gathered_refs = outr4.reshape(B, PADK, 4)[:, :KEEP, :3]
    gathered_scores = outs[:, :KEEP]
    gathered_sources = outsrc[:, :KEEP]
    gathered_logits = outl[:, :KEEP]
    return (gathered_queries, gathered_refs, gathered_scores, gathered_sources, gathered_logits)


# speedup vs baseline: 1.5909x; 1.0771x over previous
import functools

import jax
import jax.numpy as jnp
from jax import lax
from jax.experimental import pallas as pl
from jax.experimental.pallas import tpu as pltpu
from jax.experimental.pallas import tpu_sc as plsc

B = 8
N_LIDAR, N_PROP, N_GLOB = 16384, 8192, 8192
N_TOT = N_LIDAR + N_PROP + N_GLOB
D = 128
KEEP = 1000
CHUNK = 4096
N_CH_L = N_LIDAR // CHUNK
N_CH_P = N_PROP // CHUNK
N_CH_G = N_GLOB // CHUNK
N_CH = N_CH_L + N_CH_P + N_CH_G


def _score_body(emb_ref, w1_ref, b1_ref, w2_ref, b2_ref,
                lq_ref, ls_ref, pq_ref, ps_ref, gq_ref, gs_ref,
                ks_ref, kl_ref, as_ref):
    g = pl.program_id(1)
    is_l = g < N_CH_L
    is_p = jnp.logical_and(g >= N_CH_L, g < N_CH_L + N_CH_P)
    q = jnp.where(is_l, lq_ref[0, 0], jnp.where(is_p, pq_ref[0, 0], gq_ref[0, 0]))
    s = jnp.where(is_l, ls_ref[0, 0, 0], jnp.where(is_p, ps_ref[0, 0, 0], gs_ref[0, 0, 0]))
    e = jnp.where(is_l, emb_ref[0:1, :], jnp.where(is_p, emb_ref[1:2, :], emb_ref[2:3, :]))
    aq = q + e
    feat = jnp.concatenate([aq, s[:, None]], axis=1)
    h = jnp.maximum(jnp.dot(feat, w1_ref[...], preferred_element_type=jnp.float32)
                    + b1_ref[0:1, :], 0.0)
    logits = lax.dot_general(
        w2_ref[...], h, dimension_numbers=(((0,), (1,)), ((), ())),
        preferred_element_type=jnp.float32)[0] + b2_ref[0, 0]
    ks_ref[0, 0, 0, :] = logits + s
    kl_ref[0, 0, 0, :] = logits
    as_ref[0, 0, 0, :] = s


@functools.partial(jax.jit, static_argnames=("interpret",))
def _score_call(lq, ls, pq, ps, gq, gs, emb, w1, b1, w2, b2, interpret=False):
    ls3 = ls.reshape(B, N_CH_L, 1, CHUNK)
    ps3 = ps.reshape(B, N_CH_P, 1, CHUNK)
    gs3 = gs.reshape(B, N_CH_G, 1, CHUNK)
    grid = (B, N_CH)

    def qmap(lo, hi):
        return lambda b, g: (b, jnp.clip(g - lo, 0, hi - lo - 1), 0, 0)

    def smap(lo, hi):
        return lambda b, g: (b, jnp.clip(g - lo, 0, hi - lo - 1), 0, 0)

    out = pl.pallas_call(
        _score_body,
        grid=grid,
        in_specs=[
            pl.BlockSpec((3, D), lambda b, g: (0, 0)),
            pl.BlockSpec((D + 1, D), lambda b, g: (0, 0)),
            pl.BlockSpec((1, D), lambda b, g: (0, 0)),
            pl.BlockSpec((D, 1), lambda b, g: (0, 0)),
            pl.BlockSpec((1, 1), lambda b, g: (0, 0)),
            pl.BlockSpec((1, 1, CHUNK, D), qmap(0, N_CH_L)),
            pl.BlockSpec((1, 1, 1, CHUNK), smap(0, N_CH_L)),
            pl.BlockSpec((1, 1, CHUNK, D), qmap(N_CH_L, N_CH_L + N_CH_P)),
            pl.BlockSpec((1, 1, 1, CHUNK), smap(N_CH_L, N_CH_L + N_CH_P)),
            pl.BlockSpec((1, 1, CHUNK, D), qmap(N_CH_L + N_CH_P, N_CH)),
            pl.BlockSpec((1, 1, 1, CHUNK), smap(N_CH_L + N_CH_P, N_CH)),
        ],
        out_specs=[
            pl.BlockSpec((1, 1, 1, CHUNK), lambda b, g: (b, g, 0, 0)),
            pl.BlockSpec((1, 1, 1, CHUNK), lambda b, g: (b, g, 0, 0)),
            pl.BlockSpec((1, 1, 1, CHUNK), lambda b, g: (b, g, 0, 0)),
        ],
        out_shape=[jax.ShapeDtypeStruct((B, N_CH, 1, CHUNK), jnp.float32)] * 3,
        interpret=interpret,
    )(emb, w1, b1.reshape(1, D), w2, b2.reshape(1, 1),
      lq.reshape(B, N_CH_L, CHUNK, D), ls3,
      pq.reshape(B, N_CH_P, CHUNK, D), ps3,
      gq.reshape(B, N_CH_G, CHUNK, D), gs3)
    ks, kl, asc = (o.reshape(B, N_TOT) for o in out)
    return ks, kl, asc


N_VREG = N_TOT // 16
PADK = 1024


def _u32(x):
    return x.astype(jnp.uint32)


def _vperm(x, perm):
    return lax.gather(
        x, perm[:, None],
        lax.GatherDimensionNumbers(offset_dims=(), collapsed_slice_dims=(0,),
                                   start_index_map=(0,)),
        (1,), mode=lax.GatherScatterMode.PROMISE_IN_BOUNDS)


def _topk_body(ks_hbm, asc_hbm, kl_hbm, out_hbm, outs_hbm, outl_hbm,
               outsrc_hbm, key_v, asc_v, kl_v, hist_v, gtk_v, gti_v, eqi_v,
               srtk_v, srti_v, gs_v, gl_v, gsrc_v, sem):
    c = lax.axis_index("c")
    s = lax.axis_index("s")
    wid = s * 2 + c

    @pl.when(wid < B)
    def _run():
        b = wid
        lanes = lax.iota(jnp.int32, 16)
        pltpu.sync_copy(ks_hbm.at[b], key_v)
        pltpu.sync_copy(asc_hbm.at[b], asc_v)
        pltpu.sync_copy(kl_hbm.at[b], kl_v)

        def _clear_hist(i, _):
            hist_v[pl.ds(i * 16, 16)] = jnp.zeros((16,), jnp.int32)
            return 0

        ones = jnp.ones((16,), jnp.int32)

        lax.fori_loop(0, 512, _clear_hist, 0)

        def _p1(i4, _):
            for u4 in range(4):
                i = i4 * 4 + u4
                x = key_v[pl.ds(i * 16, 16)]
                u = x.astype(jnp.uint32)
                neg = u >> 31
                m = (jnp.uint32(0) - neg) | jnp.uint32(0x80000000)
                k = u ^ m
                key_v[pl.ds(i * 16, 16)] = k.astype(jnp.int32)
                d = (k >> 24).astype(jnp.int32)
                plsc.addupdate_scatter(hist_v, [d * 32 + (u4 % 2) * 16 + lanes],
                                       ones)
            return 0

        lax.fori_loop(0, N_VREG // 4, _p1, 0)

        def _scan_bins(krem):
            def bscan(i, carry):
                sel, above, cum, found = carry
                bin_ = 255 - i
                cnt = (jnp.sum(hist_v[pl.ds(bin_ * 32, 16)])
                       + jnp.sum(hist_v[pl.ds(bin_ * 32 + 16, 16)]))
                newcum = cum + cnt
                hit = jnp.logical_and(jnp.logical_not(found), newcum >= krem)
                sel = jnp.where(hit, bin_, sel)
                above = jnp.where(hit, cum, above)
                return sel, above, newcum, jnp.logical_or(found, hit)

            sel, above, _, _ = lax.fori_loop(
                0, 256, bscan, (jnp.int32(0), jnp.int32(0), jnp.int32(0),
                                jnp.bool_(False)))
            return sel, above

        krem = jnp.int32(KEEP)
        sel, above = _scan_bins(krem)
        prefix = _u32(sel)
        krem = krem - above

        for shift in (16, 8, 0):
            lax.fori_loop(0, 512, _clear_hist, 0)
            pfx = prefix

            def _pp(i4, _, shift=shift, pfx=pfx):
                for u4 in range(4):
                    i = i4 * 4 + u4
                    k = key_v[pl.ds(i * 16, 16)].astype(jnp.uint32)
                    msk = (k >> (shift + 8)) == pfx
                    d = ((k >> shift) & jnp.uint32(0xFF)).astype(jnp.int32)
                    plsc.addupdate_scatter(hist_v,
                                           [d * 32 + (u4 % 2) * 16 + lanes],
                                           ones, mask=msk)
                return 0

            lax.fori_loop(0, N_VREG // 4, _pp, 0)
            sel, above = _scan_bins(krem)
            prefix = (prefix << 8) | _u32(sel)
            krem = krem - above

        t = prefix

        def _comp(i4, carry):
            pg, pe = carry
            for u4 in range(4):
                i = i4 * 4 + u4
                k = key_v[pl.ds(i * 16, 16)].astype(jnp.uint32)
                idxv = i * 16 + lanes
                m_gt = k > t
                m_eq = k == t
                plsc.store_compressed(gtk_v.at[pl.ds(pg, 16)],
                                      k.astype(jnp.int32), mask=m_gt)
                plsc.store_compressed(gti_v.at[pl.ds(pg, 16)], idxv, mask=m_gt)

                @pl.when(pe < PADK)
                def _():
                    plsc.store_compressed(eqi_v.at[pl.ds(pe, 16)], idxv,
                                          mask=m_eq)

                pg = pg + jnp.sum(m_gt.astype(jnp.int32))
                pe = pe + jnp.sum(m_eq.astype(jnp.int32))
            return pg, pe

        n_gt, _ = lax.fori_loop(0, N_VREG // 4, _comp,
                                (jnp.int32(0), jnp.int32(0)))

        def _init(i, _):
            srtk_v[pl.ds(i * 16, 16)] = jnp.zeros((16,), jnp.uint32)
            srti_v[pl.ds(i * 16, 16)] = jnp.full((16,), 0x7FFFFFFF, jnp.int32)
            return 0

        lax.fori_loop(0, PADK // 16, _init, 0)

        def _cgt(i, _):
            pos = i * 16 + lanes
            m = pos < n_gt
            kk = gtk_v[pl.ds(i * 16, 16)].astype(jnp.uint32)
            ii = gti_v[pl.ds(i * 16, 16)]
            ok = srtk_v[pl.ds(i * 16, 16)]
            oi = srti_v[pl.ds(i * 16, 16)]
            srtk_v[pl.ds(i * 16, 16)] = jnp.where(m, kk, ok)
            srti_v[pl.ds(i * 16, 16)] = jnp.where(m, ii, oi)
            return 0

        lax.fori_loop(0, PADK // 16, _cgt, 0)

        def _ceq(j, _):
            jpos = j * 16 + lanes
            m = (n_gt + jpos) < KEEP
            e = eqi_v[pl.ds(j * 16, 16)]
            base = n_gt + j * 16
            ok = srtk_v[pl.ds(base, 16)]
            oi = srti_v[pl.ds(base, 16)]
            srtk_v[pl.ds(base, 16)] = jnp.where(m, jnp.full((16,), 1, jnp.uint32) * t, ok)
            srti_v[pl.ds(base, 16)] = jnp.where(m, e, oi)
            return 0

        lax.fori_loop(0, (KEEP + 15) // 16, _ceq, 0)

        perm_base = lanes

        def _lex_ge(ka, ia, kb, ib):
            return jnp.logical_or(
                ka > kb, jnp.logical_and(ka == kb, ia < ib))

        for size in (2, 4, 8, 16, 32, 64, 128, 256, 512, 1024):
            stride = size // 2
            while stride >= 16:
                w = stride // 16

                def _pair(p, _, w=w, size=size):
                    va = ((p & ~(w - 1)) << 1) | (p & (w - 1))
                    vb = va + w
                    dsc = ((va * 16) & size) == 0
                    ka = srtk_v[pl.ds(va * 16, 16)]
                    ia = srti_v[pl.ds(va * 16, 16)]
                    kb = srtk_v[pl.ds(vb * 16, 16)]
                    ib = srti_v[pl.ds(vb * 16, 16)]
                    ge = _lex_ge(ka, ia, kb, ib)
                    m = jnp.where(dsc, ge, jnp.logical_not(ge))
                    srtk_v[pl.ds(va * 16, 16)] = jnp.where(m, ka, kb)
                    srti_v[pl.ds(va * 16, 16)] = jnp.where(m, ia, ib)
                    srtk_v[pl.ds(vb * 16, 16)] = jnp.where(m, kb, ka)
                    srti_v[pl.ds(vb * 16, 16)] = jnp.where(m, ib, ia)
                    return 0

                lax.fori_loop(0, PADK // 32, _pair, 0)
                stride //= 2
            while stride >= 1:
                perm = perm_base ^ stride

                def _intra(v, _, stride=stride, size=size, perm=perm):
                    kk = srtk_v[pl.ds(v * 16, 16)]
                    ii = srti_v[pl.ds(v * 16, 16)]
                    kp = _vperm(kk, perm)
                    ip = _vperm(ii, perm)
                    low = (lanes & stride) == 0
                    dsc = ((v * 16 + lanes) & size) == 0
                    ge = _lex_ge(kk, ii, kp, ip)
                    cond = ge == (low == dsc)
                    srtk_v[pl.ds(v * 16, 16)] = jnp.where(cond, kk, kp)
                    srti_v[pl.ds(v * 16, 16)] = jnp.where(cond, ii, ip)
                    return 0

                lax.fori_loop(0, PADK // 16, _intra, 0)
                stride //= 2

        srti_v[pl.ds(KEEP, 16)] = lanes * 8
        srti_v[pl.ds(PADK - 16, 16)] = (lanes + 16) * 8

        def _gout(i, _):
            sidx = srti_v[pl.ds(i * 16, 16)]
            gs_v[pl.ds(i * 16, 16)] = plsc.load_gather(asc_v, [sidx])
            gl_v[pl.ds(i * 16, 16)] = plsc.load_gather(kl_v, [sidx])
            gsrc_v[pl.ds(i * 16, 16)] = (
                (sidx >= N_LIDAR).astype(jnp.int32)
                + (sidx >= N_LIDAR + N_PROP).astype(jnp.int32))
            return 0

        lax.fori_loop(0, PADK // 16, _gout, 0)
        pltpu.sync_copy(srti_v, out_hbm.at[b])
        pltpu.sync_copy(gs_v, outs_hbm.at[b])
        pltpu.sync_copy(gl_v, outl_hbm.at[b])
        pltpu.sync_copy(gsrc_v, outsrc_hbm.at[b])


@jax.jit
def _topk_call(ks, asc, kl):
    ksb = lax.bitcast_convert_type(ks, jnp.int32)
    mesh = plsc.VectorSubcoreMesh(core_axis_name="c", subcore_axis_name="s")
    f = pl.kernel(
        _topk_body,
        out_type=[
            jax.ShapeDtypeStruct((B, PADK), jnp.int32),
            jax.ShapeDtypeStruct((B, PADK), jnp.float32),
            jax.ShapeDtypeStruct((B, PADK), jnp.float32),
            jax.ShapeDtypeStruct((B, PADK), jnp.int32),
        ],
        mesh=mesh,
        compiler_params=pltpu.CompilerParams(needs_layout_passes=False),
        scratch_types=[
            pltpu.VMEM((N_TOT,), jnp.int32),
            pltpu.VMEM((N_TOT,), jnp.float32),
            pltpu.VMEM((N_TOT,), jnp.float32),
            pltpu.VMEM((256 * 32,), jnp.int32),
            pltpu.VMEM((PADK + 16,), jnp.int32),
            pltpu.VMEM((PADK + 16,), jnp.int32),
            pltpu.VMEM((PADK + 16,), jnp.int32),
            pltpu.VMEM((PADK,), jnp.uint32),
            pltpu.VMEM((PADK,), jnp.int32),
            pltpu.VMEM((PADK,), jnp.float32),
            pltpu.VMEM((PADK,), jnp.float32),
            pltpu.VMEM((PADK,), jnp.int32),
            pltpu.SemaphoreType.DMA,
        ],
    )
    return f(ksb, asc, kl)


ROWS_W = PADK // 4
HALF = 128


def _gather_body(top_hbm, lq_hbm, pq_hbm, gq_hbm, refs128_hbm, emb_hbm,
                 outq_hbm, outr4_hbm,
                 idx_v, rid_v, off_v, rid2_v, rows2_v, pos2_v,
                 r0_v, r1_v, r2_v, q0_v, q1_v, q2_v,
                 rrow_v, rbig_v, qrow_v, emb_v, sem):
    c = lax.axis_index("c")
    s = lax.axis_index("s")
    wid = s * 2 + c
    b = wid // 4
    part = wid % 4
    lanes = lax.iota(jnp.int32, 16)
    obase = b * PADK + part * ROWS_W

    pltpu.sync_copy(top_hbm.at[b, pl.ds(part * ROWS_W, ROWS_W)], idx_v)
    pltpu.sync_copy(emb_hbm, emb_v)

    def _dflt(j, _):
        safe = b * N_PROP + j * 16 + lanes
        dump = jnp.full((16,), b * PADK + PADK - 1, jnp.int32)
        r0_v[pl.ds(j * 16, 16)] = safe
        r1_v[pl.ds(j * 16, 16)] = safe
        r2_v[pl.ds(j * 16, 16)] = safe
        q0_v[pl.ds(j * 16, 16)] = dump
        q1_v[pl.ds(j * 16, 16)] = dump
        q2_v[pl.ds(j * 16, 16)] = dump
        return 0

    lax.fori_loop(0, ROWS_W // 16 + 1, _dflt, 0)

    def _split(j, carry):
        p0, p1, p2 = carry
        ix = idx_v[pl.ds(j * 16, 16)]
        rid_v[pl.ds(j * 16, 16)] = b * (N_TOT // 32) + (ix >> 5)
        off_v[pl.ds(j * 16, 16)] = (ix & 31) * 4
        pos = obase + j * 16 + lanes
        m0 = ix < N_LIDAR
        m2 = ix >= N_LIDAR + N_PROP
        m1 = jnp.logical_and(jnp.logical_not(m0), jnp.logical_not(m2))
        plsc.store_compressed(r0_v.at[pl.ds(p0, 16)], b * N_LIDAR + ix, mask=m0)
        plsc.store_compressed(q0_v.at[pl.ds(p0, 16)], pos, mask=m0)
        plsc.store_compressed(r1_v.at[pl.ds(p1, 16)], b * N_PROP + (ix - N_LIDAR),
                              mask=m1)
        plsc.store_compressed(q1_v.at[pl.ds(p1, 16)], pos, mask=m1)
        plsc.store_compressed(r2_v.at[pl.ds(p2, 16)],
                              b * N_GLOB + (ix - (N_LIDAR + N_PROP)), mask=m2)
        plsc.store_compressed(q2_v.at[pl.ds(p2, 16)], pos, mask=m2)
        p0 = p0 + jnp.sum(m0.astype(jnp.int32))
        p1 = p1 + jnp.sum(m1.astype(jnp.int32))
        p2 = p2 + jnp.sum(m2.astype(jnp.int32))
        return p0, p1, p2

    p0, p1, p2 = lax.fori_loop(0, ROWS_W // 16, _split,
                               (jnp.int32(0), jnp.int32(0), jnp.int32(0)))

    for s3, (tab, rv, qv, cnt) in enumerate(
            ((lq_hbm, r0_v, q0_v, p0), (pq_hbm, r1_v, q1_v, p1),
             (gq_hbm, r2_v, q2_v, p2))):
        for h in range(2):
            for cc in range(HALF // 16):
                rows2_v[h, pl.ds(cc * 16, 16)] = rv[pl.ds(h * HALF + cc * 16, 16)]
                pos2_v[h, pl.ds(cc * 16, 16)] = qv[pl.ds(h * HALF + cc * 16, 16)]
        for h in range(2):

            def _do_half(h=h, s3=s3):
                pltpu.async_copy(tab.at[rows2_v.at[h]], qrow_v, sem).wait()

                def _embadd(r, _, s3=s3):
                    for c8 in range(D // 16):
                        e = emb_v[pl.ds(s3 * D + c8 * 16, 16)]
                        qrow_v[r, pl.ds(c8 * 16, 16)] = (
                            qrow_v[r, pl.ds(c8 * 16, 16)] + e)
                    return 0

                lax.fori_loop(0, HALF, _embadd, 0)
                pltpu.async_copy(qrow_v, outq_hbm.at[pos2_v.at[h]], sem).wait()

            if h == 0:
                _do_half()
            else:
                pl.when(cnt > HALF)(_do_half)

    for h in range(2):
        for cc in range(HALF // 16):
            rid2_v[h, pl.ds(cc * 16, 16)] = rid_v[pl.ds(h * HALF + cc * 16, 16)]
    for h in range(2):
        pltpu.async_copy(refs128_hbm.at[rid2_v.at[h]], rbig_v, sem).wait()

        def _rext(j, _, h=h):
            rloc = j * 16 + lanes
            off = off_v[pl.ds(h * HALF + j * 16, 16)]
            for ccc in range(4):
                vals = plsc.load_gather(rbig_v, [rloc, off + ccc])
                plsc.store_scatter(rrow_v, [rloc * 4 + ccc], vals)
            return 0

        lax.fori_loop(0, HALF // 16, _rext, 0)
        pltpu.sync_copy(rrow_v, outr4_hbm.at[pl.ds((obase + h * HALF) * 4,
                                                   HALF * 4)])


@jax.jit
def _gather_call(top_idx, lq, pq, gq, refs128, embf):
    mesh = plsc.VectorSubcoreMesh(core_axis_name="c", subcore_axis_name="s")
    f = pl.kernel(
        _gather_body,
        out_type=[
            jax.ShapeDtypeStruct((B * PADK, D), jnp.float32),
            jax.ShapeDtypeStruct((B * PADK * 4,), jnp.float32),
        ],
        mesh=mesh,
        compiler_params=pltpu.CompilerParams(needs_layout_passes=False),
        scratch_types=[
            pltpu.VMEM((ROWS_W,), jnp.int32),
            pltpu.VMEM((ROWS_W,), jnp.int32),
            pltpu.VMEM((ROWS_W,), jnp.int32),
            pltpu.VMEM((2, HALF), jnp.int32),
            pltpu.VMEM((2, HALF), jnp.int32),
            pltpu.VMEM((2, HALF), jnp.int32),
            pltpu.VMEM((ROWS_W + 16,), jnp.int32),
            pltpu.VMEM((ROWS_W + 16,), jnp.int32),
            pltpu.VMEM((ROWS_W + 16,), jnp.int32),
            pltpu.VMEM((ROWS_W + 16,), jnp.int32),
            pltpu.VMEM((ROWS_W + 16,), jnp.int32),
            pltpu.VMEM((ROWS_W + 16,), jnp.int32),
            pltpu.VMEM((HALF * 4,), jnp.float32),
            pltpu.VMEM((HALF, D), jnp.float32),
            pltpu.VMEM((HALF, D), jnp.float32),
            pltpu.VMEM((3 * D,), jnp.float32),
            pltpu.SemaphoreType.DMA,
        ],
    )
    return f(top_idx, lq, pq, gq, refs128, embf)


def kernel(lidar_queries, lidar_refs, lidar_scores,
           proposal_queries, proposal_refs, proposal_scores,
           global_queries, global_refs, global_scores,
           source_embeddings, W1, b1, W2, b2):
    ks, kl, asc = _score_call(lidar_queries, lidar_scores,
                              proposal_queries, proposal_scores,
                              global_queries, global_scores,
                              source_embeddings, W1, b1, W2, b2)
    top_idx, outs, outl, outsrc = _topk_call(ks, asc, kl)
    refs128 = jnp.pad(
        jnp.concatenate([lidar_refs, proposal_refs, global_refs], axis=1),
        ((0, 0), (0, 0), (0, 1))).reshape(B * N_TOT // 32, 128)
    outq, outr4 = _gather_call(
        top_idx,
        lidar_queries.reshape(B * N_LIDAR, D),
        proposal_queries.reshape(B * N_PROP, D),
        global_queries.reshape(B * N_GLOB, D),
        refs128, source_embeddings.reshape(3 * D))
    gathered_queries = outq.reshape(B, PADK, D)[:, :KEEP]
    gathered_refs = outr4.reshape(B, PADK, 4)[:, :KEEP, :3]
    gathered_scores = outs[:, :KEEP]
    gathered_sources = outsrc[:, :KEEP]
    gathered_logits = outl[:, :KEEP]
    return (gathered_queries, gathered_refs, gathered_scores, gathered_sources, gathered_logits)


# speedup vs baseline: 1.6203x; 1.0185x over previous
import functools

import jax
import jax.numpy as jnp
from jax import lax
from jax.experimental import pallas as pl
from jax.experimental.pallas import tpu as pltpu
from jax.experimental.pallas import tpu_sc as plsc

B = 8
N_LIDAR, N_PROP, N_GLOB = 16384, 8192, 8192
N_TOT = N_LIDAR + N_PROP + N_GLOB
D = 128
KEEP = 1000
CHUNK = 8192
N_CH_L = N_LIDAR // CHUNK
N_CH_P = N_PROP // CHUNK
N_CH_G = N_GLOB // CHUNK
N_CH = N_CH_L + N_CH_P + N_CH_G


def _score_body(emb_ref, w1_ref, b1_ref, w2_ref, b2_ref,
                lq_ref, ls_ref, pq_ref, ps_ref, gq_ref, gs_ref,
                ks_ref, kl_ref, as_ref):
    g = pl.program_id(1)
    is_l = g < N_CH_L
    is_p = jnp.logical_and(g >= N_CH_L, g < N_CH_L + N_CH_P)
    q = jnp.where(is_l, lq_ref[0, 0], jnp.where(is_p, pq_ref[0, 0], gq_ref[0, 0]))
    s = jnp.where(is_l, ls_ref[0, 0, 0], jnp.where(is_p, ps_ref[0, 0, 0], gs_ref[0, 0, 0]))
    e = jnp.where(is_l, emb_ref[0:1, :], jnp.where(is_p, emb_ref[1:2, :], emb_ref[2:3, :]))
    aq = q + e
    feat = jnp.concatenate([aq, s[:, None]], axis=1)
    h = jnp.maximum(jnp.dot(feat, w1_ref[...], preferred_element_type=jnp.float32)
                    + b1_ref[0:1, :], 0.0)
    logits = lax.dot_general(
        w2_ref[...], h, dimension_numbers=(((0,), (1,)), ((), ())),
        preferred_element_type=jnp.float32)[0] + b2_ref[0, 0]
    ks_ref[0, 0, 0, :] = logits + s
    kl_ref[0, 0, 0, :] = logits
    as_ref[0, 0, 0, :] = s


@functools.partial(jax.jit, static_argnames=("interpret",))
def _score_call(lq, ls, pq, ps, gq, gs, emb, w1, b1, w2, b2, interpret=False):
    ls3 = ls.reshape(B, N_CH_L, 1, CHUNK)
    ps3 = ps.reshape(B, N_CH_P, 1, CHUNK)
    gs3 = gs.reshape(B, N_CH_G, 1, CHUNK)
    grid = (B, N_CH)

    def qmap(lo, hi):
        return lambda b, g: (b, jnp.clip(g - lo, 0, hi - lo - 1), 0, 0)

    def smap(lo, hi):
        return lambda b, g: (b, jnp.clip(g - lo, 0, hi - lo - 1), 0, 0)

    out = pl.pallas_call(
        _score_body,
        grid=grid,
        in_specs=[
            pl.BlockSpec((3, D), lambda b, g: (0, 0)),
            pl.BlockSpec((D + 1, D), lambda b, g: (0, 0)),
            pl.BlockSpec((1, D), lambda b, g: (0, 0)),
            pl.BlockSpec((D, 1), lambda b, g: (0, 0)),
            pl.BlockSpec((1, 1), lambda b, g: (0, 0)),
            pl.BlockSpec((1, 1, CHUNK, D), qmap(0, N_CH_L)),
            pl.BlockSpec((1, 1, 1, CHUNK), smap(0, N_CH_L)),
            pl.BlockSpec((1, 1, CHUNK, D), qmap(N_CH_L, N_CH_L + N_CH_P)),
            pl.BlockSpec((1, 1, 1, CHUNK), smap(N_CH_L, N_CH_L + N_CH_P)),
            pl.BlockSpec((1, 1, CHUNK, D), qmap(N_CH_L + N_CH_P, N_CH)),
            pl.BlockSpec((1, 1, 1, CHUNK), smap(N_CH_L + N_CH_P, N_CH)),
        ],
        out_specs=[
            pl.BlockSpec((1, 1, 1, CHUNK), lambda b, g: (b, g, 0, 0)),
            pl.BlockSpec((1, 1, 1, CHUNK), lambda b, g: (b, g, 0, 0)),
            pl.BlockSpec((1, 1, 1, CHUNK), lambda b, g: (b, g, 0, 0)),
        ],
        out_shape=[jax.ShapeDtypeStruct((B, N_CH, 1, CHUNK), jnp.float32)] * 3,
        interpret=interpret,
    )(emb, w1, b1.reshape(1, D), w2, b2.reshape(1, 1),
      lq.reshape(B, N_CH_L, CHUNK, D), ls3,
      pq.reshape(B, N_CH_P, CHUNK, D), ps3,
      gq.reshape(B, N_CH_G, CHUNK, D), gs3)
    ks, kl, asc = (o.reshape(B, N_TOT) for o in out)
    return ks, kl, asc


N_VREG = N_TOT // 16
PADK = 1024


def _u32(x):
    return x.astype(jnp.uint32)


def _vperm(x, perm):
    return lax.gather(
        x, perm[:, None],
        lax.GatherDimensionNumbers(offset_dims=(), collapsed_slice_dims=(0,),
                                   start_index_map=(0,)),
        (1,), mode=lax.GatherScatterMode.PROMISE_IN_BOUNDS)


def _topk_body(ks_hbm, asc_hbm, kl_hbm, out_hbm, outs_hbm, outl_hbm,
               outsrc_hbm, key_v, asc_v, kl_v, hist_v, gtk_v, gti_v, eqi_v,
               srtk_v, srti_v, gs_v, gl_v, gsrc_v, sem):
    c = lax.axis_index("c")
    s = lax.axis_index("s")
    wid = s * 2 + c

    @pl.when(wid < B)
    def _run():
        b = wid
        lanes = lax.iota(jnp.int32, 16)
        pltpu.sync_copy(ks_hbm.at[b], key_v)
        pltpu.sync_copy(asc_hbm.at[b], asc_v)
        pltpu.sync_copy(kl_hbm.at[b], kl_v)

        def _clear_hist(i, _):
            hist_v[pl.ds(i * 16, 16)] = jnp.zeros((16,), jnp.int32)
            return 0

        ones = jnp.ones((16,), jnp.int32)

        lax.fori_loop(0, 512, _clear_hist, 0)

        def _p1(i4, _):
            for u4 in range(4):
                i = i4 * 4 + u4
                x = key_v[pl.ds(i * 16, 16)]
                u = x.astype(jnp.uint32)
                neg = u >> 31
                m = (jnp.uint32(0) - neg) | jnp.uint32(0x80000000)
                k = u ^ m
                key_v[pl.ds(i * 16, 16)] = k.astype(jnp.int32)
                d = (k >> 24).astype(jnp.int32)
                plsc.addupdate_scatter(hist_v, [d * 32 + (u4 % 2) * 16 + lanes],
                                       ones)
            return 0

        lax.fori_loop(0, N_VREG // 4, _p1, 0)

        def _scan_bins(krem):
            def bscan(i, carry):
                sel, above, cum, found = carry
                bin_ = 255 - i
                cnt = (jnp.sum(hist_v[pl.ds(bin_ * 32, 16)])
                       + jnp.sum(hist_v[pl.ds(bin_ * 32 + 16, 16)]))
                newcum = cum + cnt
                hit = jnp.logical_and(jnp.logical_not(found), newcum >= krem)
                sel = jnp.where(hit, bin_, sel)
                above = jnp.where(hit, cum, above)
                return sel, above, newcum, jnp.logical_or(found, hit)

            sel, above, _, _ = lax.fori_loop(
                0, 256, bscan, (jnp.int32(0), jnp.int32(0), jnp.int32(0),
                                jnp.bool_(False)))
            return sel, above

        krem = jnp.int32(KEEP)
        sel, above = _scan_bins(krem)
        prefix = _u32(sel)
        krem = krem - above

        for shift in (16, 8, 0):
            lax.fori_loop(0, 512, _clear_hist, 0)
            pfx = prefix

            def _pp(i4, _, shift=shift, pfx=pfx):
                for u4 in range(4):
                    i = i4 * 4 + u4
                    k = key_v[pl.ds(i * 16, 16)].astype(jnp.uint32)
                    msk = (k >> (shift + 8)) == pfx
                    d = ((k >> shift) & jnp.uint32(0xFF)).astype(jnp.int32)
                    plsc.addupdate_scatter(hist_v,
                                           [d * 32 + (u4 % 2) * 16 + lanes],
                                           ones, mask=msk)
                return 0

            lax.fori_loop(0, N_VREG // 4, _pp, 0)
            sel, above = _scan_bins(krem)
            prefix = (prefix << 8) | _u32(sel)
            krem = krem - above

        t = prefix

        def _comp(i4, carry):
            pg, pe = carry
            for u4 in range(4):
                i = i4 * 4 + u4
                k = key_v[pl.ds(i * 16, 16)].astype(jnp.uint32)
                idxv = i * 16 + lanes
                m_gt = k > t
                m_eq = k == t
                plsc.store_compressed(gtk_v.at[pl.ds(pg, 16)],
                                      k.astype(jnp.int32), mask=m_gt)
                plsc.store_compressed(gti_v.at[pl.ds(pg, 16)], idxv, mask=m_gt)

                @pl.when(pe < PADK)
                def _():
                    plsc.store_compressed(eqi_v.at[pl.ds(pe, 16)], idxv,
                                          mask=m_eq)

                pg = pg + jnp.sum(m_gt.astype(jnp.int32))
                pe = pe + jnp.sum(m_eq.astype(jnp.int32))
            return pg, pe

        n_gt, _ = lax.fori_loop(0, N_VREG // 4, _comp,
                                (jnp.int32(0), jnp.int32(0)))

        def _init(i, _):
            srtk_v[pl.ds(i * 16, 16)] = jnp.zeros((16,), jnp.uint32)
            srti_v[pl.ds(i * 16, 16)] = jnp.full((16,), 0x7FFFFFFF, jnp.int32)
            return 0

        lax.fori_loop(0, PADK // 16, _init, 0)

        def _cgt(i, _):
            pos = i * 16 + lanes
            m = pos < n_gt
            kk = gtk_v[pl.ds(i * 16, 16)].astype(jnp.uint32)
            ii = gti_v[pl.ds(i * 16, 16)]
            ok = srtk_v[pl.ds(i * 16, 16)]
            oi = srti_v[pl.ds(i * 16, 16)]
            srtk_v[pl.ds(i * 16, 16)] = jnp.where(m, kk, ok)
            srti_v[pl.ds(i * 16, 16)] = jnp.where(m, ii, oi)
            return 0

        lax.fori_loop(0, PADK // 16, _cgt, 0)

        def _ceq(j, _):
            jpos = j * 16 + lanes
            m = (n_gt + jpos) < KEEP
            e = eqi_v[pl.ds(j * 16, 16)]
            base = n_gt + j * 16
            ok = srtk_v[pl.ds(base, 16)]
            oi = srti_v[pl.ds(base, 16)]
            srtk_v[pl.ds(base, 16)] = jnp.where(m, jnp.full((16,), 1, jnp.uint32) * t, ok)
            srti_v[pl.ds(base, 16)] = jnp.where(m, e, oi)
            return 0

        lax.fori_loop(0, (KEEP + 15) // 16, _ceq, 0)

        perm_base = lanes

        def _lex_ge(ka, ia, kb, ib):
            return jnp.logical_or(
                ka > kb, jnp.logical_and(ka == kb, ia < ib))

        for size in (2, 4, 8, 16, 32, 64, 128, 256, 512, 1024):
            stride = size // 2
            while stride >= 16:
                w = stride // 16

                def _pair(p, _, w=w, size=size):
                    va = ((p & ~(w - 1)) << 1) | (p & (w - 1))
                    vb = va + w
                    dsc = ((va * 16) & size) == 0
                    ka = srtk_v[pl.ds(va * 16, 16)]
                    ia = srti_v[pl.ds(va * 16, 16)]
                    kb = srtk_v[pl.ds(vb * 16, 16)]
                    ib = srti_v[pl.ds(vb * 16, 16)]
                    ge = _lex_ge(ka, ia, kb, ib)
                    m = jnp.where(dsc, ge, jnp.logical_not(ge))
                    srtk_v[pl.ds(va * 16, 16)] = jnp.where(m, ka, kb)
                    srti_v[pl.ds(va * 16, 16)] = jnp.where(m, ia, ib)
                    srtk_v[pl.ds(vb * 16, 16)] = jnp.where(m, kb, ka)
                    srti_v[pl.ds(vb * 16, 16)] = jnp.where(m, ib, ia)
                    return 0

                lax.fori_loop(0, PADK // 32, _pair, 0)
                stride //= 2
            while stride >= 1:
                perm = perm_base ^ stride

                def _intra(v, _, stride=stride, size=size, perm=perm):
                    kk = srtk_v[pl.ds(v * 16, 16)]
                    ii = srti_v[pl.ds(v * 16, 16)]
                    kp = _vperm(kk, perm)
                    ip = _vperm(ii, perm)
                    low = (lanes & stride) == 0
                    dsc = ((v * 16 + lanes) & size) == 0
                    ge = _lex_ge(kk, ii, kp, ip)
                    cond = ge == (low == dsc)
                    srtk_v[pl.ds(v * 16, 16)] = jnp.where(cond, kk, kp)
                    srti_v[pl.ds(v * 16, 16)] = jnp.where(cond, ii, ip)
                    return 0

                lax.fori_loop(0, PADK // 16, _intra, 0)
                stride //= 2

        srti_v[pl.ds(KEEP, 16)] = lanes * 8
        srti_v[pl.ds(PADK - 16, 16)] = (lanes + 16) * 8

        def _gout(i, _):
            sidx = srti_v[pl.ds(i * 16, 16)]
            gs_v[pl.ds(i * 16, 16)] = plsc.load_gather(asc_v, [sidx])
            gl_v[pl.ds(i * 16, 16)] = plsc.load_gather(kl_v, [sidx])
            gsrc_v[pl.ds(i * 16, 16)] = (
                (sidx >= N_LIDAR).astype(jnp.int32)
                + (sidx >= N_LIDAR + N_PROP).astype(jnp.int32))
            return 0

        lax.fori_loop(0, PADK // 16, _gout, 0)
        pltpu.sync_copy(srti_v, out_hbm.at[b])
        pltpu.sync_copy(gs_v, outs_hbm.at[b])
        pltpu.sync_copy(gl_v, outl_hbm.at[b])
        pltpu.sync_copy(gsrc_v, outsrc_hbm.at[b])


@jax.jit
def _topk_call(ks, asc, kl):
    ksb = lax.bitcast_convert_type(ks, jnp.int32)
    mesh = plsc.VectorSubcoreMesh(core_axis_name="c", subcore_axis_name="s")
    f = pl.kernel(
        _topk_body,
        out_type=[
            jax.ShapeDtypeStruct((B, PADK), jnp.int32),
            jax.ShapeDtypeStruct((B, PADK), jnp.float32),
            jax.ShapeDtypeStruct((B, PADK), jnp.float32),
            jax.ShapeDtypeStruct((B, PADK), jnp.int32),
        ],
        mesh=mesh,
        compiler_params=pltpu.CompilerParams(needs_layout_passes=False),
        scratch_types=[
            pltpu.VMEM((N_TOT,), jnp.int32),
            pltpu.VMEM((N_TOT,), jnp.float32),
            pltpu.VMEM((N_TOT,), jnp.float32),
            pltpu.VMEM((256 * 32,), jnp.int32),
            pltpu.VMEM((PADK + 16,), jnp.int32),
            pltpu.VMEM((PADK + 16,), jnp.int32),
            pltpu.VMEM((PADK + 16,), jnp.int32),
            pltpu.VMEM((PADK,), jnp.uint32),
            pltpu.VMEM((PADK,), jnp.int32),
            pltpu.VMEM((PADK,), jnp.float32),
            pltpu.VMEM((PADK,), jnp.float32),
            pltpu.VMEM((PADK,), jnp.int32),
            pltpu.SemaphoreType.DMA,
        ],
    )
    return f(ksb, asc, kl)


ROWS_W = PADK // 4
HALF = 128


def _gather_body(top_hbm, lq_hbm, pq_hbm, gq_hbm, refs128_hbm, emb_hbm,
                 outq_hbm, outr4_hbm,
                 idx_v, rid_v, off_v, rid2_v, rows2_v, pos2_v,
                 r0_v, r1_v, r2_v, q0_v, q1_v, q2_v,
                 rrow_v, rbig_v, qrow_v, emb_v, sem):
    c = lax.axis_index("c")
    s = lax.axis_index("s")
    wid = s * 2 + c
    b = wid // 4
    part = wid % 4
    lanes = lax.iota(jnp.int32, 16)
    obase = b * PADK + part * ROWS_W

    pltpu.sync_copy(top_hbm.at[b, pl.ds(part * ROWS_W, ROWS_W)], idx_v)
    pltpu.sync_copy(emb_hbm, emb_v)

    def _dflt(j, _):
        safe = b * N_PROP + j * 16 + lanes
        dump = jnp.full((16,), b * PADK + PADK - 1, jnp.int32)
        r0_v[pl.ds(j * 16, 16)] = safe
        r1_v[pl.ds(j * 16, 16)] = safe
        r2_v[pl.ds(j * 16, 16)] = safe
        q0_v[pl.ds(j * 16, 16)] = dump
        q1_v[pl.ds(j * 16, 16)] = dump
        q2_v[pl.ds(j * 16, 16)] = dump
        return 0

    lax.fori_loop(0, ROWS_W // 16 + 1, _dflt, 0)

    def _split(j, carry):
        p0, p1, p2 = carry
        ix = idx_v[pl.ds(j * 16, 16)]
        rid_v[pl.ds(j * 16, 16)] = b * (N_TOT // 32) + (ix >> 5)
        off_v[pl.ds(j * 16, 16)] = (ix & 31) * 4
        pos = obase + j * 16 + lanes
        m0 = ix < N_LIDAR
        m2 = ix >= N_LIDAR + N_PROP
        m1 = jnp.logical_and(jnp.logical_not(m0), jnp.logical_not(m2))
        plsc.store_compressed(r0_v.at[pl.ds(p0, 16)], b * N_LIDAR + ix, mask=m0)
        plsc.store_compressed(q0_v.at[pl.ds(p0, 16)], pos, mask=m0)
        plsc.store_compressed(r1_v.at[pl.ds(p1, 16)], b * N_PROP + (ix - N_LIDAR),
                              mask=m1)
        plsc.store_compressed(q1_v.at[pl.ds(p1, 16)], pos, mask=m1)
        plsc.store_compressed(r2_v.at[pl.ds(p2, 16)],
                              b * N_GLOB + (ix - (N_LIDAR + N_PROP)), mask=m2)
        plsc.store_compressed(q2_v.at[pl.ds(p2, 16)], pos, mask=m2)
        p0 = p0 + jnp.sum(m0.astype(jnp.int32))
        p1 = p1 + jnp.sum(m1.astype(jnp.int32))
        p2 = p2 + jnp.sum(m2.astype(jnp.int32))
        return p0, p1, p2

    p0, p1, p2 = lax.fori_loop(0, ROWS_W // 16, _split,
                               (jnp.int32(0), jnp.int32(0), jnp.int32(0)))

    for s3, (tab, rv, qv, cnt) in enumerate(
            ((lq_hbm, r0_v, q0_v, p0), (pq_hbm, r1_v, q1_v, p1),
             (gq_hbm, r2_v, q2_v, p2))):
        for h in range(2):
            for cc in range(HALF // 16):
                rows2_v[h, pl.ds(cc * 16, 16)] = rv[pl.ds(h * HALF + cc * 16, 16)]
                pos2_v[h, pl.ds(cc * 16, 16)] = qv[pl.ds(h * HALF + cc * 16, 16)]
        for h in range(2):

            def _do_half(h=h, s3=s3):
                pltpu.async_copy(tab.at[rows2_v.at[h]], qrow_v, sem).wait()

                def _embadd(r, _, s3=s3):
                    for c8 in range(D // 16):
                        e = emb_v[pl.ds(s3 * D + c8 * 16, 16)]
                        qrow_v[r, pl.ds(c8 * 16, 16)] = (
                            qrow_v[r, pl.ds(c8 * 16, 16)] + e)
                    return 0

                lax.fori_loop(0, HALF, _embadd, 0)
                pltpu.async_copy(qrow_v, outq_hbm.at[pos2_v.at[h]], sem).wait()

            if h == 0:
                _do_half()
            else:
                pl.when(cnt > HALF)(_do_half)

    for h in range(2):
        for cc in range(HALF // 16):
            rid2_v[h, pl.ds(cc * 16, 16)] = rid_v[pl.ds(h * HALF + cc * 16, 16)]
    for h in range(2):
        pltpu.async_copy(refs128_hbm.at[rid2_v.at[h]], rbig_v, sem).wait()

        def _rext(j, _, h=h):
            rloc = j * 16 + lanes
            off = off_v[pl.ds(h * HALF + j * 16, 16)]
            for ccc in range(4):
                vals = plsc.load_gather(rbig_v, [rloc, off + ccc])
                plsc.store_scatter(rrow_v, [rloc * 4 + ccc], vals)
            return 0

        lax.fori_loop(0, HALF // 16, _rext, 0)
        pltpu.sync_copy(rrow_v, outr4_hbm.at[pl.ds((obase + h * HALF) * 4,
                                                   HALF * 4)])


@jax.jit
def _gather_call(top_idx, lq, pq, gq, refs128, embf):
    mesh = plsc.VectorSubcoreMesh(core_axis_name="c", subcore_axis_name="s")
    f = pl.kernel(
        _gather_body,
        out_type=[
            jax.ShapeDtypeStruct((B * PADK, D), jnp.float32),
            jax.ShapeDtypeStruct((B * PADK * 4,), jnp.float32),
        ],
        mesh=mesh,
        compiler_params=pltpu.CompilerParams(needs_layout_passes=False),
        scratch_types=[
            pltpu.VMEM((ROWS_W,), jnp.int32),
            pltpu.VMEM((ROWS_W,), jnp.int32),
            pltpu.VMEM((ROWS_W,), jnp.int32),
            pltpu.VMEM((2, HALF), jnp.int32),
            pltpu.VMEM((2, HALF), jnp.int32),
            pltpu.VMEM((2, HALF), jnp.int32),
            pltpu.VMEM((ROWS_W + 16,), jnp.int32),
            pltpu.VMEM((ROWS_W + 16,), jnp.int32),
            pltpu.VMEM((ROWS_W + 16,), jnp.int32),
            pltpu.VMEM((ROWS_W + 16,), jnp.int32),
            pltpu.VMEM((ROWS_W + 16,), jnp.int32),
            pltpu.VMEM((ROWS_W + 16,), jnp.int32),
            pltpu.VMEM((HALF * 4,), jnp.float32),
            pltpu.VMEM((HALF, D), jnp.float32),
            pltpu.VMEM((HALF, D), jnp.float32),
            pltpu.VMEM((3 * D,), jnp.float32),
            pltpu.SemaphoreType.DMA,
        ],
    )
    return f(top_idx, lq, pq, gq, refs128, embf)


def kernel(lidar_queries, lidar_refs, lidar_scores,
           proposal_queries, proposal_refs, proposal_scores,
           global_queries, global_refs, global_scores,
           source_embeddings, W1, b1, W2, b2):
    ks, kl, asc = _score_call(lidar_queries, lidar_scores,
                              proposal_queries, proposal_scores,
                              global_queries, global_scores,
                              source_embeddings, W1, b1, W2, b2)
    top_idx, outs, outl, outsrc = _topk_call(ks, asc, kl)
    refs128 = jnp.pad(
        jnp.concatenate([lidar_refs, proposal_refs, global_refs], axis=1),
        ((0, 0), (0, 0), (0, 1))).reshape(B * N_TOT // 32, 128)
    outq, outr4 = _gather_call(
        top_idx,
        lidar_queries.reshape(B * N_LIDAR, D),
        proposal_queries.reshape(B * N_PROP, D),
        global_queries.reshape(B * N_GLOB, D),
        refs128, source_embeddings.reshape(3 * D))
    gathered_queries = outq.reshape(B, PADK, D)[:, :KEEP]
    gathered_refs = outr4.reshape(B, PADK, 4)[:, :KEEP, :3]
    gathered_scores = outs[:, :KEEP]
    gathered_sources = outsrc[:, :KEEP]
    gathered_logits = outl[:, :KEEP]
    return (gathered_queries, gathered_refs, gathered_scores, gathered_sources, gathered_logits)


# speedup vs baseline: 1.6219x; 1.0010x over previous
import jax
import jax.numpy as jnp
from jax import lax
from jax.experimental import pallas as pl
from jax.experimental.pallas import tpu as pltpu
from jax.experimental.pallas import tpu_sc as plsc

B = 8
N_LIDAR, N_PROP, N_GLOB = 16384, 8192, 8192
N_TOT = N_LIDAR + N_PROP + N_GLOB
D = 128
KEEP = 1000
CHUNK = 8192
N_CH_L = N_LIDAR // CHUNK
N_CH_P = N_PROP // CHUNK
N_CH_G = N_GLOB // CHUNK
N_CH = N_CH_L + N_CH_P + N_CH_G


def _score_body(emb_ref, w1_ref, b1_ref, w2_ref, b2_ref,
                lq_ref, ls_ref, pq_ref, ps_ref, gq_ref, gs_ref,
                ks_ref, kl_ref, as_ref):
    g = pl.program_id(1)
    is_l = g < N_CH_L
    is_p = jnp.logical_and(g >= N_CH_L, g < N_CH_L + N_CH_P)
    q = jnp.where(is_l, lq_ref[0, 0], jnp.where(is_p, pq_ref[0, 0], gq_ref[0, 0]))
    s = jnp.where(is_l, ls_ref[0, 0, 0], jnp.where(is_p, ps_ref[0, 0, 0], gs_ref[0, 0, 0]))
    e = jnp.where(is_l, emb_ref[0:1, :], jnp.where(is_p, emb_ref[1:2, :], emb_ref[2:3, :]))
    aq = q + e
    feat = jnp.concatenate([aq, s[:, None]], axis=1)
    h = jnp.maximum(jnp.dot(feat, w1_ref[...], preferred_element_type=jnp.float32)
                    + b1_ref[0:1, :], 0.0)
    logits = lax.dot_general(
        w2_ref[...], h, dimension_numbers=(((0,), (1,)), ((), ())),
        preferred_element_type=jnp.float32)[0] + b2_ref[0, 0]
    ks_ref[0, 0, 0, :] = logits + s
    kl_ref[0, 0, 0, :] = logits
    as_ref[0, 0, 0, :] = s


@jax.jit
def _score_call(lq, ls, pq, ps, gq, gs, emb, w1, b1, w2, b2):
    ls3 = ls.reshape(B, N_CH_L, 1, CHUNK)
    ps3 = ps.reshape(B, N_CH_P, 1, CHUNK)
    gs3 = gs.reshape(B, N_CH_G, 1, CHUNK)
    grid = (B, N_CH)

    def qmap(lo, hi):
        return lambda b, g: (b, jnp.clip(g - lo, 0, hi - lo - 1), 0, 0)

    def smap(lo, hi):
        return lambda b, g: (b, jnp.clip(g - lo, 0, hi - lo - 1), 0, 0)

    out = pl.pallas_call(
        _score_body,
        grid=grid,
        in_specs=[
            pl.BlockSpec((3, D), lambda b, g: (0, 0)),
            pl.BlockSpec((D + 1, D), lambda b, g: (0, 0)),
            pl.BlockSpec((1, D), lambda b, g: (0, 0)),
            pl.BlockSpec((D, 1), lambda b, g: (0, 0)),
            pl.BlockSpec((1, 1), lambda b, g: (0, 0)),
            pl.BlockSpec((1, 1, CHUNK, D), qmap(0, N_CH_L)),
            pl.BlockSpec((1, 1, 1, CHUNK), smap(0, N_CH_L)),
            pl.BlockSpec((1, 1, CHUNK, D), qmap(N_CH_L, N_CH_L + N_CH_P)),
            pl.BlockSpec((1, 1, 1, CHUNK), smap(N_CH_L, N_CH_L + N_CH_P)),
            pl.BlockSpec((1, 1, CHUNK, D), qmap(N_CH_L + N_CH_P, N_CH)),
            pl.BlockSpec((1, 1, 1, CHUNK), smap(N_CH_L + N_CH_P, N_CH)),
        ],
        out_specs=[
            pl.BlockSpec((1, 1, 1, CHUNK), lambda b, g: (b, g, 0, 0)),
            pl.BlockSpec((1, 1, 1, CHUNK), lambda b, g: (b, g, 0, 0)),
            pl.BlockSpec((1, 1, 1, CHUNK), lambda b, g: (b, g, 0, 0)),
        ],
        out_shape=[jax.ShapeDtypeStruct((B, N_CH, 1, CHUNK), jnp.float32)] * 3,
    )(emb, w1, b1.reshape(1, D), w2, b2.reshape(1, 1),
      lq.reshape(B, N_CH_L, CHUNK, D), ls3,
      pq.reshape(B, N_CH_P, CHUNK, D), ps3,
      gq.reshape(B, N_CH_G, CHUNK, D), gs3)
    ks, kl, asc = (o.reshape(B, N_TOT) for o in out)
    return ks, kl, asc


N_VREG = N_TOT // 16
PADK = 1024


def _u32(x):
    return x.astype(jnp.uint32)


def _vperm(x, perm):
    return lax.gather(
        x, perm[:, None],
        lax.GatherDimensionNumbers(offset_dims=(), collapsed_slice_dims=(0,),
                                   start_index_map=(0,)),
        (1,), mode=lax.GatherScatterMode.PROMISE_IN_BOUNDS)


def _topk_body(ks_hbm, asc_hbm, kl_hbm, out_hbm, outs_hbm, outl_hbm,
               outsrc_hbm, key_v, asc_v, kl_v, hist_v, gtk_v, gti_v, eqi_v,
               srtk_v, srti_v, gs_v, gl_v, gsrc_v, sem):
    c = lax.axis_index("c")
    s = lax.axis_index("s")
    wid = s * 2 + c

    @pl.when(wid < B)
    def _run():
        b = wid
        lanes = lax.iota(jnp.int32, 16)
        pltpu.sync_copy(ks_hbm.at[b], key_v)
        pltpu.sync_copy(asc_hbm.at[b], asc_v)
        pltpu.sync_copy(kl_hbm.at[b], kl_v)

        def _clear_hist(i, _):
            hist_v[pl.ds(i * 16, 16)] = jnp.zeros((16,), jnp.int32)
            return 0

        ones = jnp.ones((16,), jnp.int32)

        lax.fori_loop(0, 512, _clear_hist, 0)

        def _p1(i4, _):
            for u4 in range(4):
                i = i4 * 4 + u4
                x = key_v[pl.ds(i * 16, 16)]
                u = x.astype(jnp.uint32)
                neg = u >> 31
                m = (jnp.uint32(0) - neg) | jnp.uint32(0x80000000)
                k = u ^ m
                key_v[pl.ds(i * 16, 16)] = k.astype(jnp.int32)
                d = (k >> 24).astype(jnp.int32)
                plsc.addupdate_scatter(hist_v, [d * 32 + (u4 % 2) * 16 + lanes],
                                       ones)
            return 0

        lax.fori_loop(0, N_VREG // 4, _p1, 0)

        def _scan_bins(krem):
            def bscan(i, carry):
                sel, above, cum, found = carry
                bin_ = 255 - i
                cnt = (jnp.sum(hist_v[pl.ds(bin_ * 32, 16)])
                       + jnp.sum(hist_v[pl.ds(bin_ * 32 + 16, 16)]))
                newcum = cum + cnt
                hit = jnp.logical_and(jnp.logical_not(found), newcum >= krem)
                sel = jnp.where(hit, bin_, sel)
                above = jnp.where(hit, cum, above)
                return sel, above, newcum, jnp.logical_or(found, hit)

            sel, above, _, _ = lax.fori_loop(
                0, 256, bscan, (jnp.int32(0), jnp.int32(0), jnp.int32(0),
                                jnp.bool_(False)))
            return sel, above

        krem = jnp.int32(KEEP)
        sel, above = _scan_bins(krem)
        prefix = _u32(sel)
        krem = krem - above

        for shift in (16, 8, 0):
            lax.fori_loop(0, 512, _clear_hist, 0)
            pfx = prefix

            def _pp(i4, _, shift=shift, pfx=pfx):
                for u4 in range(4):
                    i = i4 * 4 + u4
                    k = key_v[pl.ds(i * 16, 16)].astype(jnp.uint32)
                    msk = (k >> (shift + 8)) == pfx
                    d = ((k >> shift) & jnp.uint32(0xFF)).astype(jnp.int32)
                    plsc.addupdate_scatter(hist_v,
                                           [d * 32 + (u4 % 2) * 16 + lanes],
                                           ones, mask=msk)
                return 0

            lax.fori_loop(0, N_VREG // 4, _pp, 0)
            sel, above = _scan_bins(krem)
            prefix = (prefix << 8) | _u32(sel)
            krem = krem - above

        t = prefix

        def _comp(i4, carry):
            pg, pe = carry
            for u4 in range(4):
                i = i4 * 4 + u4
                k = key_v[pl.ds(i * 16, 16)].astype(jnp.uint32)
                idxv = i * 16 + lanes
                m_gt = k > t
                m_eq = k == t
                plsc.store_compressed(gtk_v.at[pl.ds(pg, 16)],
                                      k.astype(jnp.int32), mask=m_gt)
                plsc.store_compressed(gti_v.at[pl.ds(pg, 16)], idxv, mask=m_gt)

                @pl.when(pe < PADK)
                def _():
                    plsc.store_compressed(eqi_v.at[pl.ds(pe, 16)], idxv,
                                          mask=m_eq)

                pg = pg + jnp.sum(m_gt.astype(jnp.int32))
                pe = pe + jnp.sum(m_eq.astype(jnp.int32))
            return pg, pe

        n_gt, _ = lax.fori_loop(0, N_VREG // 4, _comp,
                                (jnp.int32(0), jnp.int32(0)))

        def _init(i, _):
            srtk_v[pl.ds(i * 16, 16)] = jnp.zeros((16,), jnp.uint32)
            srti_v[pl.ds(i * 16, 16)] = jnp.full((16,), 0x7FFFFFFF, jnp.int32)
            return 0

        lax.fori_loop(0, PADK // 16, _init, 0)

        def _cgt(i, _):
            pos = i * 16 + lanes
            m = pos < n_gt
            kk = gtk_v[pl.ds(i * 16, 16)].astype(jnp.uint32)
            ii = gti_v[pl.ds(i * 16, 16)]
            ok = srtk_v[pl.ds(i * 16, 16)]
            oi = srti_v[pl.ds(i * 16, 16)]
            srtk_v[pl.ds(i * 16, 16)] = jnp.where(m, kk, ok)
            srti_v[pl.ds(i * 16, 16)] = jnp.where(m, ii, oi)
            return 0

        lax.fori_loop(0, PADK // 16, _cgt, 0)

        def _ceq(j, _):
            jpos = j * 16 + lanes
            m = (n_gt + jpos) < KEEP
            e = eqi_v[pl.ds(j * 16, 16)]
            base = n_gt + j * 16
            ok = srtk_v[pl.ds(base, 16)]
            oi = srti_v[pl.ds(base, 16)]
            srtk_v[pl.ds(base, 16)] = jnp.where(m, jnp.full((16,), 1, jnp.uint32) * t, ok)
            srti_v[pl.ds(base, 16)] = jnp.where(m, e, oi)
            return 0

        lax.fori_loop(0, (KEEP + 15) // 16, _ceq, 0)

        perm_base = lanes

        def _lex_ge(ka, ia, kb, ib):
            return jnp.logical_or(
                ka > kb, jnp.logical_and(ka == kb, ia < ib))

        for size in (2, 4, 8, 16, 32, 64, 128, 256, 512, 1024):
            stride = size // 2
            while stride >= 16:
                w = stride // 16

                def _pair(p, _, w=w, size=size):
                    va = ((p & ~(w - 1)) << 1) | (p & (w - 1))
                    vb = va + w
                    dsc = ((va * 16) & size) == 0
                    ka = srtk_v[pl.ds(va * 16, 16)]
                    ia = srti_v[pl.ds(va * 16, 16)]
                    kb = srtk_v[pl.ds(vb * 16, 16)]
                    ib = srti_v[pl.ds(vb * 16, 16)]
                    ge = _lex_ge(ka, ia, kb, ib)
                    m = jnp.where(dsc, ge, jnp.logical_not(ge))
                    srtk_v[pl.ds(va * 16, 16)] = jnp.where(m, ka, kb)
                    srti_v[pl.ds(va * 16, 16)] = jnp.where(m, ia, ib)
                    srtk_v[pl.ds(vb * 16, 16)] = jnp.where(m, kb, ka)
                    srti_v[pl.ds(vb * 16, 16)] = jnp.where(m, ib, ia)
                    return 0

                lax.fori_loop(0, PADK // 32, _pair, 0)
                stride //= 2
            while stride >= 1:
                perm = perm_base ^ stride

                def _intra(v, _, stride=stride, size=size, perm=perm):
                    kk = srtk_v[pl.ds(v * 16, 16)]
                    ii = srti_v[pl.ds(v * 16, 16)]
                    kp = _vperm(kk, perm)
                    ip = _vperm(ii, perm)
                    low = (lanes & stride) == 0
                    dsc = ((v * 16 + lanes) & size) == 0
                    ge = _lex_ge(kk, ii, kp, ip)
                    cond = ge == (low == dsc)
                    srtk_v[pl.ds(v * 16, 16)] = jnp.where(cond, kk, kp)
                    srti_v[pl.ds(v * 16, 16)] = jnp.where(cond, ii, ip)
                    return 0

                lax.fori_loop(0, PADK // 16, _intra, 0)
                stride //= 2

        srti_v[pl.ds(KEEP, 16)] = lanes * 8
        srti_v[pl.ds(PADK - 16, 16)] = (lanes + 16) * 8

        def _gout(i, _):
            sidx = srti_v[pl.ds(i * 16, 16)]
            gs_v[pl.ds(i * 16, 16)] = plsc.load_gather(asc_v, [sidx])
            gl_v[pl.ds(i * 16, 16)] = plsc.load_gather(kl_v, [sidx])
            gsrc_v[pl.ds(i * 16, 16)] = (
                (sidx >= N_LIDAR).astype(jnp.int32)
                + (sidx >= N_LIDAR + N_PROP).astype(jnp.int32))
            return 0

        lax.fori_loop(0, PADK // 16, _gout, 0)
        pltpu.sync_copy(srti_v, out_hbm.at[b])
        pltpu.sync_copy(gs_v, outs_hbm.at[b])
        pltpu.sync_copy(gl_v, outl_hbm.at[b])
        pltpu.sync_copy(gsrc_v, outsrc_hbm.at[b])


@jax.jit
def _topk_call(ks, asc, kl):
    ksb = lax.bitcast_convert_type(ks, jnp.int32)
    mesh = plsc.VectorSubcoreMesh(core_axis_name="c", subcore_axis_name="s")
    f = pl.kernel(
        _topk_body,
        out_type=[
            jax.ShapeDtypeStruct((B, PADK), jnp.int32),
            jax.ShapeDtypeStruct((B, PADK), jnp.float32),
            jax.ShapeDtypeStruct((B, PADK), jnp.float32),
            jax.ShapeDtypeStruct((B, PADK), jnp.int32),
        ],
        mesh=mesh,
        compiler_params=pltpu.CompilerParams(needs_layout_passes=False),
        scratch_types=[
            pltpu.VMEM((N_TOT,), jnp.int32),
            pltpu.VMEM((N_TOT,), jnp.float32),
            pltpu.VMEM((N_TOT,), jnp.float32),
            pltpu.VMEM((256 * 32,), jnp.int32),
            pltpu.VMEM((PADK + 16,), jnp.int32),
            pltpu.VMEM((PADK + 16,), jnp.int32),
            pltpu.VMEM((PADK + 16,), jnp.int32),
            pltpu.VMEM((PADK,), jnp.uint32),
            pltpu.VMEM((PADK,), jnp.int32),
            pltpu.VMEM((PADK,), jnp.float32),
            pltpu.VMEM((PADK,), jnp.float32),
            pltpu.VMEM((PADK,), jnp.int32),
            pltpu.SemaphoreType.DMA,
        ],
    )
    return f(ksb, asc, kl)


ROWS_W = PADK // 4
HALF = 128


def _gather_body(top_hbm, lq_hbm, pq_hbm, gq_hbm, refs128_hbm, emb_hbm,
                 outq_hbm, outr4_hbm,
                 idx_v, rid_v, off_v, rid2_v, rows2_v, pos2_v,
                 r0_v, r1_v, r2_v, q0_v, q1_v, q2_v,
                 rrow_v, rbig_v, qrow_v, emb_v, sem):
    c = lax.axis_index("c")
    s = lax.axis_index("s")
    wid = s * 2 + c
    b = wid // 4
    part = wid % 4
    lanes = lax.iota(jnp.int32, 16)
    obase = b * PADK + part * ROWS_W

    pltpu.sync_copy(top_hbm.at[b, pl.ds(part * ROWS_W, ROWS_W)], idx_v)
    pltpu.sync_copy(emb_hbm, emb_v)

    def _dflt(j, _):
        safe = b * N_PROP + j * 16 + lanes
        dump = jnp.full((16,), b * PADK + PADK - 1, jnp.int32)
        r0_v[pl.ds(j * 16, 16)] = safe
        r1_v[pl.ds(j * 16, 16)] = safe
        r2_v[pl.ds(j * 16, 16)] = safe
        q0_v[pl.ds(j * 16, 16)] = dump
        q1_v[pl.ds(j * 16, 16)] = dump
        q2_v[pl.ds(j * 16, 16)] = dump
        return 0

    lax.fori_loop(0, ROWS_W // 16 + 1, _dflt, 0)

    def _split(j, carry):
        p0, p1, p2 = carry
        ix = idx_v[pl.ds(j * 16, 16)]
        rid_v[pl.ds(j * 16, 16)] = b * (N_TOT // 32) + (ix >> 5)
        off_v[pl.ds(j * 16, 16)] = (ix & 31) * 4
        pos = obase + j * 16 + lanes
        m0 = ix < N_LIDAR
        m2 = ix >= N_LIDAR + N_PROP
        m1 = jnp.logical_and(jnp.logical_not(m0), jnp.logical_not(m2))
        plsc.store_compressed(r0_v.at[pl.ds(p0, 16)], b * N_LIDAR + ix, mask=m0)
        plsc.store_compressed(q0_v.at[pl.ds(p0, 16)], pos, mask=m0)
        plsc.store_compressed(r1_v.at[pl.ds(p1, 16)], b * N_PROP + (ix - N_LIDAR),
                              mask=m1)
        plsc.store_compressed(q1_v.at[pl.ds(p1, 16)], pos, mask=m1)
        plsc.store_compressed(r2_v.at[pl.ds(p2, 16)],
                              b * N_GLOB + (ix - (N_LIDAR + N_PROP)), mask=m2)
        plsc.store_compressed(q2_v.at[pl.ds(p2, 16)], pos, mask=m2)
        p0 = p0 + jnp.sum(m0.astype(jnp.int32))
        p1 = p1 + jnp.sum(m1.astype(jnp.int32))
        p2 = p2 + jnp.sum(m2.astype(jnp.int32))
        return p0, p1, p2

    p0, p1, p2 = lax.fori_loop(0, ROWS_W // 16, _split,
                               (jnp.int32(0), jnp.int32(0), jnp.int32(0)))

    for s3, (tab, rv, qv, cnt) in enumerate(
            ((lq_hbm, r0_v, q0_v, p0), (pq_hbm, r1_v, q1_v, p1),
             (gq_hbm, r2_v, q2_v, p2))):
        for h in range(2):
            for cc in range(HALF // 16):
                rows2_v[h, pl.ds(cc * 16, 16)] = rv[pl.ds(h * HALF + cc * 16, 16)]
                pos2_v[h, pl.ds(cc * 16, 16)] = qv[pl.ds(h * HALF + cc * 16, 16)]
        for h in range(2):

            def _do_half(h=h, s3=s3):
                pltpu.async_copy(tab.at[rows2_v.at[h]], qrow_v, sem).wait()

                def _embadd(r, _, s3=s3):
                    for c8 in range(D // 16):
                        e = emb_v[pl.ds(s3 * D + c8 * 16, 16)]
                        qrow_v[r, pl.ds(c8 * 16, 16)] = (
                            qrow_v[r, pl.ds(c8 * 16, 16)] + e)
                    return 0

                lax.fori_loop(0, HALF, _embadd, 0)
                pltpu.async_copy(qrow_v, outq_hbm.at[pos2_v.at[h]], sem).wait()

            if h == 0:
                _do_half()
            else:
                pl.when(cnt > HALF)(_do_half)

    for h in range(2):
        for cc in range(HALF // 16):
            rid2_v[h, pl.ds(cc * 16, 16)] = rid_v[pl.ds(h * HALF + cc * 16, 16)]
    for h in range(2):
        pltpu.async_copy(refs128_hbm.at[rid2_v.at[h]], rbig_v, sem).wait()

        def _rext(j, _, h=h):
            rloc = j * 16 + lanes
            off = off_v[pl.ds(h * HALF + j * 16, 16)]
            for ccc in range(4):
                vals = plsc.load_gather(rbig_v, [rloc, off + ccc])
                plsc.store_scatter(rrow_v, [rloc * 4 + ccc], vals)
            return 0

        lax.fori_loop(0, HALF // 16, _rext, 0)
        pltpu.sync_copy(rrow_v, outr4_hbm.at[pl.ds((obase + h * HALF) * 4,
                                                   HALF * 4)])


@jax.jit
def _gather_call(top_idx, lq, pq, gq, refs128, embf):
    mesh = plsc.VectorSubcoreMesh(core_axis_name="c", subcore_axis_name="s")
    f = pl.kernel(
        _gather_body,
        out_type=[
            jax.ShapeDtypeStruct((B * PADK, D), jnp.float32),
            jax.ShapeDtypeStruct((B * PADK * 4,), jnp.float32),
        ],
        mesh=mesh,
        compiler_params=pltpu.CompilerParams(needs_layout_passes=False),
        scratch_types=[
            pltpu.VMEM((ROWS_W,), jnp.int32),
            pltpu.VMEM((ROWS_W,), jnp.int32),
            pltpu.VMEM((ROWS_W,), jnp.int32),
            pltpu.VMEM((2, HALF), jnp.int32),
            pltpu.VMEM((2, HALF), jnp.int32),
            pltpu.VMEM((2, HALF), jnp.int32),
            pltpu.VMEM((ROWS_W + 16,), jnp.int32),
            pltpu.VMEM((ROWS_W + 16,), jnp.int32),
            pltpu.VMEM((ROWS_W + 16,), jnp.int32),
            pltpu.VMEM((ROWS_W + 16,), jnp.int32),
            pltpu.VMEM((ROWS_W + 16,), jnp.int32),
            pltpu.VMEM((ROWS_W + 16,), jnp.int32),
            pltpu.VMEM((HALF * 4,), jnp.float32),
            pltpu.VMEM((HALF, D), jnp.float32),
            pltpu.VMEM((HALF, D), jnp.float32),
            pltpu.VMEM((3 * D,), jnp.float32),
            pltpu.SemaphoreType.DMA,
        ],
    )
    return f(top_idx, lq, pq, gq, refs128, embf)


def kernel(lidar_queries, lidar_refs, lidar_scores,
           proposal_queries, proposal_refs, proposal_scores,
           global_queries, global_refs, global_scores,
           source_embeddings, W1, b1, W2, b2):
    ks, kl, asc = _score_call(lidar_queries, lidar_scores,
                              proposal_queries, proposal_scores,
                              global_queries, global_scores,
                              source_embeddings, W1, b1, W2, b2)
    top_idx, outs, outl, outsrc = _topk_call(ks, asc, kl)
    refs128 = jnp.pad(
        jnp.concatenate([lidar_refs, proposal_refs, global_refs], axis=1),
        ((0, 0), (0, 0), (0, 1))).reshape(B * N_TOT // 32, 128)
    outq, outr4 = _gather_call(
        top_idx,
        lidar_queries.reshape(B * N_LIDAR, D),
        proposal_queries.reshape(B * N_PROP, D),
        global_queries.reshape(B * N_GLOB, D),
        refs128, source_embeddings.reshape(3 * D))
    gathered_queries = outq.reshape(B, PADK, D)[:, :KEEP]
    gathered_refs = outr4.reshape(B, PADK, 4)[:, :KEEP, :3]
    gathered_scores = outs[:, :KEEP]
    gathered_sources = outsrc[:, :KEEP]
    gathered_logits = outl[:, :KEEP]
    return (gathered_queries, gathered_refs, gathered_scores, gathered_sources, gathered_logits)
